# baseline (speedup 1.0000x reference)
import functools

import jax
import jax.numpy as jnp
from jax import lax
from jax.experimental import pallas as pl
from jax.experimental.pallas import tpu as pltpu
from jax.experimental.pallas import tpu_sc as plsc

N = 10000
E = 160000
IN_FEATS = 128
H1 = 128
H2 = 64

NC = 2
NS = 16
NW = NC * NS
LANES = 16

N_PAD = 10240
N_UPAD = 10048
R = 512
NBUF = 2

SLOW_CID = 0
K1, S1, F1 = 64, 50, 110
K2, S2, F2 = 128, 26, 54
CAP1 = K1 * F1
CAP2 = K2 * F2
E_PAD = NS * (S1 + F1) * K1
assert NS * (S2 + F2) * K2 == E_PAD

F1E = 144
F2E = 80
ZROWS = 4
RPT = N_UPAD // NS


def _elu(x):
    return jnp.where(x > 0, x, jnp.exp(jnp.minimum(x, 0.0)) - 1.0)


def _tc_pre_body(f_ref, w_ref, alr_ref, hx_ref, eler_ref, mm_ref):
    i = pl.program_id(0)
    fext = hx_ref.shape[1]
    ones_col = fext - LANES
    h = lax.dot_general(f_ref[...], w_ref[...], (((1,), (1,)), ((), ())),
                        preferred_element_type=jnp.float32)
    lane = lax.broadcasted_iota(jnp.int32, h.shape, 1)
    h = h + jnp.where(lane == ones_col, 1.0, 0.0)
    hx_ref[...] = h
    eler = lax.dot_general(alr_ref[...], h, (((1,), (1,)), ((), ())),
                           preferred_element_type=jnp.float32)
    eler_ref[...] = eler
    mblk = jnp.max(eler, axis=1, keepdims=True)

    @pl.when(i == 0)
    def _():
        mm_ref[...] = mblk

    @pl.when(i > 0)
    def _():
        mm_ref[...] = jnp.maximum(mm_ref[...], mblk)


def _tc_pre(feat_p, w1p, alr1):
    grid = (N_PAD // R,)
    return pl.pallas_call(
        _tc_pre_body,
        grid=grid,
        in_specs=[
            pl.BlockSpec((R, IN_FEATS), lambda i: (i, 0)),
            pl.BlockSpec((F1E, IN_FEATS), lambda i: (0, 0)),
            pl.BlockSpec((2, F1E), lambda i: (0, 0)),
        ],
        out_specs=[
            pl.BlockSpec((R, F1E), lambda i: (i, 0)),
            pl.BlockSpec((2, R), lambda i: (0, i)),
            pl.BlockSpec((2, 1), lambda i: (0, 0)),
        ],
        out_shape=[
            jax.ShapeDtypeStruct((N_PAD, F1E), jnp.float32),
            jax.ShapeDtypeStruct((2, N_PAD), jnp.float32),
            jax.ShapeDtypeStruct((2, 1), jnp.float32),
        ],
    )(feat_p, w1p, alr1)


def _tc_mid_body(ua_ref, ub_ref, b1_ref, w2_ref, alr_ref, hx_ref, eler_ref,
                 mm_ref):
    i = pl.program_id(0)
    u = ua_ref[...] + ub_ref[...]
    denom = jnp.maximum(u[:, H1:H1 + 1], 1e-9)
    rst = u[:, :H1] / denom + b1_ref[...]
    x = _elu(_elu(rst))
    h = lax.dot_general(x, w2_ref[...], (((1,), (1,)), ((), ())),
                        preferred_element_type=jnp.float32)
    lane = lax.broadcasted_iota(jnp.int32, h.shape, 1)
    h = h + jnp.where(lane == H2, 1.0, 0.0)
    hx_ref[...] = h
    eler = lax.dot_general(alr_ref[...], h, (((1,), (1,)), ((), ())),
                           preferred_element_type=jnp.float32)
    row = lax.broadcasted_iota(jnp.int32, eler.shape, 1) + i * R
    eler = jnp.where(row < N_UPAD, eler, -1e30)
    eler_ref[...] = eler
    mblk = jnp.max(eler, axis=1, keepdims=True)

    @pl.when(i == 0)
    def _():
        mm_ref[...] = mblk

    @pl.when(i > 0)
    def _():
        mm_ref[...] = jnp.maximum(mm_ref[...], mblk)


def _tc_mid(ua, ub, b1, w2p, alr2):
    grid = (N_PAD // R,)
    return pl.pallas_call(
        _tc_mid_body,
        grid=grid,
        in_specs=[
            pl.BlockSpec((R, F1E), lambda i: (i, 0)),
            pl.BlockSpec((R, F1E), lambda i: (i, 0)),
            pl.BlockSpec((1, H1), lambda i: (0, 0)),
            pl.BlockSpec((F2E, H1), lambda i: (0, 0)),
            pl.BlockSpec((2, F2E), lambda i: (0, 0)),
        ],
        out_specs=[
            pl.BlockSpec((R, F2E), lambda i: (i, 0)),
            pl.BlockSpec((2, R), lambda i: (0, i)),
            pl.BlockSpec((2, 1), lambda i: (0, 0)),
        ],
        out_shape=[
            jax.ShapeDtypeStruct((N_PAD, F2E), jnp.float32),
            jax.ShapeDtypeStruct((2, N_PAD), jnp.float32),
            jax.ShapeDtypeStruct((2, 1), jnp.float32),
        ],
    )(ua, ub, b1, w2p, alr2)


def _tc_final_body(ua_ref, ub_ref, b2_ref, wl_ref, bl_ref, us_ref, sp_ref,
                   scal_ref, out_ref):
    u = ua_ref[...] + ub_ref[...]
    denom = jnp.maximum(u[:, H2:H2 + 1], 1e-9)
    x = _elu(u[:, :H2] / denom + b2_ref[...])
    zt = lax.dot_general(wl_ref[...], x, (((1,), (1,)), ((), ())),
                         preferred_element_type=jnp.float32)
    zt = zt + bl_ref[...]
    sig = 1.0 / (1.0 + jnp.exp(-zt))
    alpha0 = scal_ref[0:1, 0:1]
    beta0 = scal_ref[0:1, 1:2]
    gamma0 = scal_ref[0:1, 2:3]
    dt = scal_ref[0:1, 3:4]
    beta = sig[0:1, :] * beta0
    gamma = sig[1:2, :] * gamma0
    alphas = sig[2:3, :] * alpha0
    us = us_ref[...]
    sp = sp_ref[...]
    up_out = us + (alphas - beta * us) * dt
    sp_out = sp + (beta * us - gamma * sp) * dt
    zero3 = jnp.zeros((3, up_out.shape[1]), jnp.float32)
    out_ref[...] = jnp.concatenate([up_out, sp_out, alphas, beta, gamma, zero3], 0)


def _tc_final(ua, ub, b2, wlp, blp, us, sp, scal):
    grid = (N_PAD // R,)
    return pl.pallas_call(
        _tc_final_body,
        grid=grid,
        in_specs=[
            pl.BlockSpec((R, F2E), lambda i: (i, 0)),
            pl.BlockSpec((R, F2E), lambda i: (i, 0)),
            pl.BlockSpec((1, H2), lambda i: (0, 0)),
            pl.BlockSpec((8, H2), lambda i: (0, 0)),
            pl.BlockSpec((8, 1), lambda i: (0, 0)),
            pl.BlockSpec((1, R), lambda i: (0, i)),
            pl.BlockSpec((1, R), lambda i: (0, i)),
            pl.BlockSpec((1, 4), lambda i: (0, 0)),
        ],
        out_specs=[pl.BlockSpec((8, R), lambda i: (0, i))],
        out_shape=[jax.ShapeDtypeStruct((8, N_PAD), jnp.float32)],
    )(ua, ub, b2, wlp, blp, us, sp, scal)


_SC_PARAMS = pltpu.CompilerParams(
    needs_layout_passes=False, use_tc_tiling_on_sc=False)


def _sc_attention(eler_hbm, mvec_hbm, srcp_hbm, dstp_hbm, ee_hbm,
                  el_v, er_v, src_v, dst_v, m_v, ee_v):
    cid = lax.axis_index("c")
    sid = lax.axis_index("s")
    wid = sid * NC + cid
    pltpu.sync_copy(eler_hbm.at[0], el_v)
    pltpu.sync_copy(eler_hbm.at[1], er_v)
    pltpu.sync_copy(mvec_hbm, m_v)
    pltpu.sync_copy(srcp_hbm.at[wid], src_v)
    pltpu.sync_copy(dstp_hbm.at[wid], dst_v)
    mvec = m_v[...]

    def step(t, _):
        for j in range(4):
            o = t * 64 + j * LANES
            sv = src_v[pl.ds(o, LANES)]
            dv = dst_v[pl.ds(o, LANES)]
            elg = plsc.load_gather(el_v, [sv])
            erg = plsc.load_gather(er_v, [dv])
            x = elg + erg
            e = jnp.where(x >= 0, x, x * 0.2)
            ee_v[pl.ds(o, LANES)] = jnp.exp(e - mvec)
        return 0
    lax.fori_loop(0, CAP1 // 64, step, 0)
    pltpu.sync_copy(ee_v, ee_hbm.at[wid])


def _make_sc_attention():
    mesh = plsc.VectorSubcoreMesh(core_axis_name="c", subcore_axis_name="s")
    return functools.partial(
        pl.kernel,
        out_type=[jax.ShapeDtypeStruct((NW, CAP1), jnp.float32)],
        mesh=mesh,
        compiler_params=_SC_PARAMS,
        scratch_types=[
            pltpu.VMEM((N_PAD,), jnp.float32),
            pltpu.VMEM((N_PAD,), jnp.float32),
            pltpu.VMEM((CAP1,), jnp.int32),
            pltpu.VMEM((CAP1,), jnp.int32),
            pltpu.VMEM((LANES,), jnp.float32),
            pltpu.VMEM((CAP1,), jnp.float32),
        ],
    )(_sc_attention)


def _zero_accumulator(u_sh, zero_v, sid, fext, zsem):
    zrow = jnp.zeros((LANES,), jnp.float32)

    def zb_row(zi, _):
        def zcol(qi, _):
            zero_v[zi, pl.ds(qi * LANES, LANES)] = zrow
            return 0
        lax.fori_loop(0, fext // LANES, zcol, 0)
        return 0
    lax.fori_loop(0, ZROWS, zb_row, 0)

    nz = RPT // ZROWS
    def zdma(ji, _):
        pltpu.async_copy(zero_v, u_sh.at[pl.ds(sid * RPT + ji * ZROWS, ZROWS)],
                         zsem)
        return 0
    lax.fori_loop(0, nz, zdma, 0)
    return nz


def _zero_drain(u_sh, zero_v, sid, nz, zsem):
    def zdrain(ji, _):
        pltpu.make_async_copy(
            zero_v, u_sh.at[pl.ds(sid * RPT, ZROWS)], zsem).wait()
        return 0
    lax.fori_loop(0, nz, zdrain, 0)


def _make_sc_scatter1():
    mesh = plsc.VectorSubcoreMesh(core_axis_name="c", subcore_axis_name="s")
    fext, k = F1E, K1

    def body(hx_hbm, ee_hbm, srcp_hbm, dstp_hbm, out_hbm,
             src_v, dst_v, ee_v, rows0_v, rows1_v, zero_v, u_sh,
             gsem0, gsem1, ssem0, ssem1, zsem):
        cid = lax.axis_index("c")
        sid = lax.axis_index("s")
        wid = sid * NC + cid
        cpt = jnp.where(cid == SLOW_CID, S1, F1)
        rows_bufs = (rows0_v, rows1_v)
        gsems = (gsem0, gsem1)
        ssems = (ssem0, ssem1)

        nz = _zero_accumulator(u_sh, zero_v, sid, fext, zsem)
        pltpu.sync_copy(srcp_hbm.at[wid], src_v)
        pltpu.sync_copy(dstp_hbm.at[wid], dst_v)
        pltpu.sync_copy(ee_hbm.at[wid], ee_v)
        _zero_drain(u_sh, zero_v, sid, nz, zsem)
        plsc.subcore_barrier()

        def gissue(ci, b):
            pltpu.async_copy(hx_hbm.at[src_v.at[ci]], rows_bufs[b], gsems[b])

        def gwait(b):
            pltpu.make_async_copy(hx_hbm.at[src_v.at[0]], rows_bufs[b],
                                  gsems[b]).wait()

        def sissue(ci, b):
            pltpu.async_copy(rows_bufs[b], u_sh.at[dst_v.at[ci]], ssems[b],
                             add=True)

        def swait(b):
            pltpu.make_async_copy(rows_bufs[b], u_sh.at[dst_v.at[0]],
                                  ssems[b]).wait()

        gissue(0, 0)

        def outer(g, _):
            for b in range(NBUF):
                c = g * NBUF + b
                buf = rows_bufs[b]

                @pl.when(c >= 1)
                def _():
                    swait(b ^ 1)

                @pl.when(c + 1 < cpt)
                def _():
                    gissue(c + 1, b ^ 1)
                gwait(b)

                def scale4(rg, _):
                    base = c * k + rg * 4
                    for rr in range(4):
                        ri = rg * 4 + rr
                        ee = plsc.load_gather(
                            ee_v, [jnp.full((LANES,), base + rr, jnp.int32)])
                        vals = [buf[ri, pl.ds(q * LANES, LANES)]
                                for q in range(fext // LANES)]
                        vals = [v * ee for v in vals]
                        for q in range(fext // LANES):
                            buf[ri, pl.ds(q * LANES, LANES)] = vals[q]
                    return 0
                lax.fori_loop(0, k // 4, scale4, 0)
                sissue(c, b)
            return 0
        lax.fori_loop(0, cpt // NBUF, outer, 0)
        swait(1)

        plsc.subcore_barrier()
        pltpu.sync_copy(u_sh.at[pl.ds(sid * RPT, RPT)],
                        out_hbm.at[cid, pl.ds(sid * RPT, RPT)])

    return functools.partial(
        pl.kernel,
        out_type=[jax.ShapeDtypeStruct((NC, N_PAD, fext), jnp.float32)],
        mesh=mesh,
        compiler_params=_SC_PARAMS,
        scratch_types=[
            pltpu.VMEM((F1, k), jnp.int32),
            pltpu.VMEM((F1, k), jnp.int32),
            pltpu.VMEM((CAP1,), jnp.float32),
            pltpu.VMEM((k, fext), jnp.float32),
            pltpu.VMEM((k, fext), jnp.float32),
            pltpu.VMEM((ZROWS, fext), jnp.float32),
            pltpu.VMEM_SHARED((N_UPAD, fext), jnp.float32),
            pltpu.SemaphoreType.DMA,
            pltpu.SemaphoreType.DMA,
            pltpu.SemaphoreType.DMA,
            pltpu.SemaphoreType.DMA,
            pltpu.SemaphoreType.DMA,
        ],
    )(body)


def _make_sc_edge2():
    mesh = plsc.VectorSubcoreMesh(core_axis_name="c", subcore_axis_name="s")
    fext, k = F2E, K2

    def body(hx_hbm, eler_hbm, mvec_hbm, srcp_hbm, dstp_hbm, out_hbm,
             el_v, er_v, src_v, dst_v, m_v, ee_v, rows0_v, rows1_v, zero_v,
             u_sh, gsem0, gsem1, ssem0, ssem1, zsem):
        cid = lax.axis_index("c")
        sid = lax.axis_index("s")
        wid = sid * NC + cid
        cpt = jnp.where(cid == SLOW_CID, S2, F2)
        rows_bufs = (rows0_v, rows1_v)
        gsems = (gsem0, gsem1)
        ssems = (ssem0, ssem1)

        nz = _zero_accumulator(u_sh, zero_v, sid, fext, zsem)
        pltpu.sync_copy(eler_hbm.at[0], el_v)
        pltpu.sync_copy(eler_hbm.at[1], er_v)
        pltpu.sync_copy(mvec_hbm, m_v)
        pltpu.sync_copy(srcp_hbm.at[wid], src_v)
        pltpu.sync_copy(dstp_hbm.at[wid], dst_v)
        _zero_drain(u_sh, zero_v, sid, nz, zsem)
        plsc.subcore_barrier()
        mvec = m_v[...]

        def gissue(ci, b):
            pltpu.async_copy(hx_hbm.at[src_v.at[ci]], rows_bufs[b], gsems[b])

        def gwait(b):
            pltpu.make_async_copy(hx_hbm.at[src_v.at[0]], rows_bufs[b],
                                  gsems[b]).wait()

        def sissue(ci, b):
            pltpu.async_copy(rows_bufs[b], u_sh.at[dst_v.at[ci]], ssems[b],
                             add=True)

        def swait(b):
            pltpu.make_async_copy(rows_bufs[b], u_sh.at[dst_v.at[0]],
                                  ssems[b]).wait()

        gissue(0, 0)

        def outer(g, _):
            for b in range(NBUF):
                c = g * NBUF + b
                buf = rows_bufs[b]

                @pl.when(c >= 1)
                def _():
                    swait(b ^ 1)

                @pl.when(c + 1 < cpt)
                def _():
                    gissue(c + 1, b ^ 1)

                for j in range(k // LANES):
                    sv = src_v[c, pl.ds(j * LANES, LANES)]
                    dv = dst_v[c, pl.ds(j * LANES, LANES)]
                    elg = plsc.load_gather(el_v, [sv])
                    erg = plsc.load_gather(er_v, [dv])
                    x = elg + erg
                    e = jnp.where(x >= 0, x, x * 0.2)
                    ee_v[pl.ds(j * LANES, LANES)] = jnp.exp(e - mvec)
                gwait(b)

                def scale4(rg, _):
                    for rr in range(4):
                        ri = rg * 4 + rr
                        ee = plsc.load_gather(
                            ee_v, [jnp.full((LANES,), ri, jnp.int32)])
                        vals = [buf[ri, pl.ds(q * LANES, LANES)]
                                for q in range(fext // LANES)]
                        vals = [v * ee for v in vals]
                        for q in range(fext // LANES):
                            buf[ri, pl.ds(q * LANES, LANES)] = vals[q]
                    return 0
                lax.fori_loop(0, k // 4, scale4, 0)
                sissue(c, b)
            return 0
        lax.fori_loop(0, cpt // NBUF, outer, 0)
        swait(1)

        plsc.subcore_barrier()
        pltpu.sync_copy(u_sh.at[pl.ds(sid * RPT, RPT)],
                        out_hbm.at[cid, pl.ds(sid * RPT, RPT)])

    return functools.partial(
        pl.kernel,
        out_type=[jax.ShapeDtypeStruct((NC, N_PAD, fext), jnp.float32)],
        mesh=mesh,
        compiler_params=_SC_PARAMS,
        scratch_types=[
            pltpu.VMEM((N_PAD,), jnp.float32),
            pltpu.VMEM((N_PAD,), jnp.float32),
            pltpu.VMEM((F2, k), jnp.int32),
            pltpu.VMEM((F2, k), jnp.int32),
            pltpu.VMEM((LANES,), jnp.float32),
            pltpu.VMEM((k + LANES,), jnp.float32),
            pltpu.VMEM((k, fext), jnp.float32),
            pltpu.VMEM((k, fext), jnp.float32),
            pltpu.VMEM((ZROWS, fext), jnp.float32),
            pltpu.VMEM_SHARED((N_UPAD, fext), jnp.float32),
            pltpu.SemaphoreType.DMA,
            pltpu.SemaphoreType.DMA,
            pltpu.SemaphoreType.DMA,
            pltpu.SemaphoreType.DMA,
            pltpu.SemaphoreType.DMA,
        ],
    )(body)


_sc_att_1 = _make_sc_attention()
_sc_scatter_1 = _make_sc_scatter1()
_sc_edge_2 = _make_sc_edge2()


def kernel(edge_index, feat, unsplice, splice, alpha0, beta0, gamma0, dt,
           W1, b1, al1, ar1, W2, b2, al2, ar2, Wl, bl):
    f32 = jnp.float32
    src = edge_index[0]
    dst = edge_index[1]
    pad_e = E_PAD - E
    src_p = jnp.concatenate([src, jnp.zeros((pad_e,), jnp.int32)])
    dst_p = jnp.concatenate([dst, jnp.full((pad_e,), N, jnp.int32)])

    def _distribute(slow_k, fast_k, cap):
        cids = jnp.arange(NW, dtype=jnp.int32) % NC
        counts = jnp.where(cids == SLOW_CID, slow_k, fast_k)
        off = jnp.cumsum(counts) - counts
        j = jnp.arange(cap, dtype=jnp.int32)
        idx = off[:, None] + jnp.minimum(j[None, :], counts[:, None] - 1)
        valid = j[None, :] < counts[:, None]
        sw = jnp.where(valid, src_p[idx], 0)
        dw = jnp.where(valid, dst_p[idx], N)
        return sw, dw

    srcw1, dstw1 = _distribute(S1 * K1, F1 * K1, CAP1)
    srcp1 = srcw1.reshape(NW, F1, K1)
    dstp1 = dstw1.reshape(NW, F1, K1)
    srcw2, dstw2 = _distribute(S2 * K2, F2 * K2, CAP2)
    srcp2 = srcw2.reshape(NW, F2, K2)
    dstp2 = dstw2.reshape(NW, F2, K2)

    feat_p = jnp.zeros((N_PAD, IN_FEATS), f32).at[:N].set(feat)
    w1p = jnp.zeros((F1E, IN_FEATS), f32).at[:H1].set(W1)
    alr1 = jnp.zeros((2, F1E), f32).at[0, :H1].set(al1[0]).at[1, :H1].set(ar1[0])
    w2p = jnp.zeros((F2E, H1), f32).at[:H2].set(W2)
    alr2 = jnp.zeros((2, F2E), f32).at[0, :H2].set(al2[0]).at[1, :H2].set(ar2[0])
    wlp = jnp.zeros((8, H2), f32).at[:3].set(Wl)
    blp = jnp.zeros((8, 1), f32).at[:3, 0].set(bl)
    usp = jnp.zeros((1, N_PAD), f32).at[0, :N].set(unsplice)
    spp = jnp.zeros((1, N_PAD), f32).at[0, :N].set(splice)
    scal = jnp.stack([alpha0[0], beta0[0], gamma0[0], dt[0]]).reshape(1, 4)

    hx1, eler1, mm1 = _tc_pre(feat_p, w1p, alr1)
    mvec1 = jnp.full((LANES,), mm1[0, 0] + mm1[1, 0], f32)
    (ee1,) = _sc_att_1(eler1, mvec1, srcw1, dstw1)
    (u1,) = _sc_scatter_1(hx1, ee1, srcp1, dstp1)

    hx2, eler2, mm2 = _tc_mid(u1[0], u1[1], b1.reshape(1, H1), w2p, alr2)
    mvec2 = jnp.full((LANES,), mm2[0, 0] + mm2[1, 0], f32)
    (u2,) = _sc_edge_2(hx2, eler2, mvec2, srcp2, dstp2)

    (out8,) = _tc_final(u2[0], u2[1], b2.reshape(1, H2), wlp, blp, usp, spp, scal)

    return (out8[0, :N], out8[1, :N], out8[2, :N], out8[3, :N], out8[4, :N])


# speedup vs baseline: 1.1490x; 1.1490x over previous
import functools

import jax
import jax.numpy as jnp
from jax import lax
from jax.experimental import pallas as pl
from jax.experimental.pallas import tpu as pltpu
from jax.experimental.pallas import tpu_sc as plsc

N = 10000
E = 160000
IN_FEATS = 128
H1 = 128
H2 = 64

NC = 2
NS = 16
NW = NC * NS
LANES = 16

N_PAD = 10240
N_UPAD = 10048
R = 512
NBUF = 2

SLOW_CID = 1
K1, S1, F1 = 64, 50, 110
K2, S2, F2 = 128, 26, 54
CAP1 = K1 * F1
CAP2 = K2 * F2
E_PAD = NS * (S1 + F1) * K1
assert NS * (S2 + F2) * K2 == E_PAD

F1E = 144
F2E = 80
ZROWS = 4
RPT = N_UPAD // NS


def _elu(x):
    return jnp.where(x > 0, x, jnp.exp(jnp.minimum(x, 0.0)) - 1.0)


def _tc_pre_body(f_ref, w_ref, alr_ref, hx_ref, eler_ref, mm_ref):
    i = pl.program_id(0)
    fext = hx_ref.shape[1]
    ones_col = fext - LANES
    h = lax.dot_general(f_ref[...], w_ref[...], (((1,), (1,)), ((), ())),
                        preferred_element_type=jnp.float32)
    lane = lax.broadcasted_iota(jnp.int32, h.shape, 1)
    h = h + jnp.where(lane == ones_col, 1.0, 0.0)
    hx_ref[...] = h
    eler = lax.dot_general(alr_ref[...], h, (((1,), (1,)), ((), ())),
                           preferred_element_type=jnp.float32)
    eler_ref[...] = eler
    mblk = jnp.max(eler, axis=1, keepdims=True)

    @pl.when(i == 0)
    def _():
        mm_ref[...] = mblk

    @pl.when(i > 0)
    def _():
        mm_ref[...] = jnp.maximum(mm_ref[...], mblk)


def _tc_pre(feat_p, w1p, alr1):
    grid = (N_PAD // R,)
    return pl.pallas_call(
        _tc_pre_body,
        grid=grid,
        in_specs=[
            pl.BlockSpec((R, IN_FEATS), lambda i: (i, 0)),
            pl.BlockSpec((F1E, IN_FEATS), lambda i: (0, 0)),
            pl.BlockSpec((2, F1E), lambda i: (0, 0)),
        ],
        out_specs=[
            pl.BlockSpec((R, F1E), lambda i: (i, 0)),
            pl.BlockSpec((2, R), lambda i: (0, i)),
            pl.BlockSpec((2, 1), lambda i: (0, 0)),
        ],
        out_shape=[
            jax.ShapeDtypeStruct((N_PAD, F1E), jnp.float32),
            jax.ShapeDtypeStruct((2, N_PAD), jnp.float32),
            jax.ShapeDtypeStruct((2, 1), jnp.float32),
        ],
    )(feat_p, w1p, alr1)


def _tc_mid_body(ua_ref, ub_ref, b1_ref, w2_ref, alr_ref, hx_ref, eler_ref,
                 mm_ref):
    i = pl.program_id(0)
    u = ua_ref[...] + ub_ref[...]
    denom = jnp.maximum(u[:, H1:H1 + 1], 1e-9)
    rst = u[:, :H1] / denom + b1_ref[...]
    x = _elu(_elu(rst))
    h = lax.dot_general(x, w2_ref[...], (((1,), (1,)), ((), ())),
                        preferred_element_type=jnp.float32)
    lane = lax.broadcasted_iota(jnp.int32, h.shape, 1)
    h = h + jnp.where(lane == H2, 1.0, 0.0)
    hx_ref[...] = h
    eler = lax.dot_general(alr_ref[...], h, (((1,), (1,)), ((), ())),
                           preferred_element_type=jnp.float32)
    row = lax.broadcasted_iota(jnp.int32, eler.shape, 1) + i * R
    eler = jnp.where(row < N_UPAD, eler, -1e30)
    eler_ref[...] = eler
    mblk = jnp.max(eler, axis=1, keepdims=True)

    @pl.when(i == 0)
    def _():
        mm_ref[...] = mblk

    @pl.when(i > 0)
    def _():
        mm_ref[...] = jnp.maximum(mm_ref[...], mblk)


def _tc_mid(ua, ub, b1, w2p, alr2):
    grid = (N_PAD // R,)
    return pl.pallas_call(
        _tc_mid_body,
        grid=grid,
        in_specs=[
            pl.BlockSpec((R, F1E), lambda i: (i, 0)),
            pl.BlockSpec((R, F1E), lambda i: (i, 0)),
            pl.BlockSpec((1, H1), lambda i: (0, 0)),
            pl.BlockSpec((F2E, H1), lambda i: (0, 0)),
            pl.BlockSpec((2, F2E), lambda i: (0, 0)),
        ],
        out_specs=[
            pl.BlockSpec((R, F2E), lambda i: (i, 0)),
            pl.BlockSpec((2, R), lambda i: (0, i)),
            pl.BlockSpec((2, 1), lambda i: (0, 0)),
        ],
        out_shape=[
            jax.ShapeDtypeStruct((N_PAD, F2E), jnp.float32),
            jax.ShapeDtypeStruct((2, N_PAD), jnp.float32),
            jax.ShapeDtypeStruct((2, 1), jnp.float32),
        ],
    )(ua, ub, b1, w2p, alr2)


def _tc_final_body(ua_ref, ub_ref, b2_ref, wl_ref, bl_ref, us_ref, sp_ref,
                   scal_ref, out_ref):
    u = ua_ref[...] + ub_ref[...]
    denom = jnp.maximum(u[:, H2:H2 + 1], 1e-9)
    x = _elu(u[:, :H2] / denom + b2_ref[...])
    zt = lax.dot_general(wl_ref[...], x, (((1,), (1,)), ((), ())),
                         preferred_element_type=jnp.float32)
    zt = zt + bl_ref[...]
    sig = 1.0 / (1.0 + jnp.exp(-zt))
    alpha0 = scal_ref[0:1, 0:1]
    beta0 = scal_ref[0:1, 1:2]
    gamma0 = scal_ref[0:1, 2:3]
    dt = scal_ref[0:1, 3:4]
    beta = sig[0:1, :] * beta0
    gamma = sig[1:2, :] * gamma0
    alphas = sig[2:3, :] * alpha0
    us = us_ref[...]
    sp = sp_ref[...]
    up_out = us + (alphas - beta * us) * dt
    sp_out = sp + (beta * us - gamma * sp) * dt
    zero3 = jnp.zeros((3, up_out.shape[1]), jnp.float32)
    out_ref[...] = jnp.concatenate([up_out, sp_out, alphas, beta, gamma, zero3], 0)


def _tc_final(ua, ub, b2, wlp, blp, us, sp, scal):
    grid = (N_PAD // R,)
    return pl.pallas_call(
        _tc_final_body,
        grid=grid,
        in_specs=[
            pl.BlockSpec((R, F2E), lambda i: (i, 0)),
            pl.BlockSpec((R, F2E), lambda i: (i, 0)),
            pl.BlockSpec((1, H2), lambda i: (0, 0)),
            pl.BlockSpec((8, H2), lambda i: (0, 0)),
            pl.BlockSpec((8, 1), lambda i: (0, 0)),
            pl.BlockSpec((1, R), lambda i: (0, i)),
            pl.BlockSpec((1, R), lambda i: (0, i)),
            pl.BlockSpec((1, 4), lambda i: (0, 0)),
        ],
        out_specs=[pl.BlockSpec((8, R), lambda i: (0, i))],
        out_shape=[jax.ShapeDtypeStruct((8, N_PAD), jnp.float32)],
    )(ua, ub, b2, wlp, blp, us, sp, scal)


_SC_PARAMS = pltpu.CompilerParams(
    needs_layout_passes=False, use_tc_tiling_on_sc=False)


def _sc_attention(eler_hbm, mvec_hbm, srcp_hbm, dstp_hbm, ee_hbm,
                  el_v, er_v, src_v, dst_v, m_v, ee_v):
    cid = lax.axis_index("c")
    sid = lax.axis_index("s")
    wid = sid * NC + cid
    pltpu.sync_copy(eler_hbm.at[0], el_v)
    pltpu.sync_copy(eler_hbm.at[1], er_v)
    pltpu.sync_copy(mvec_hbm, m_v)
    pltpu.sync_copy(srcp_hbm.at[wid], src_v)
    pltpu.sync_copy(dstp_hbm.at[wid], dst_v)
    mvec = m_v[...]

    def step(t, _):
        for j in range(4):
            o = t * 64 + j * LANES
            sv = src_v[pl.ds(o, LANES)]
            dv = dst_v[pl.ds(o, LANES)]
            elg = plsc.load_gather(el_v, [sv])
            erg = plsc.load_gather(er_v, [dv])
            x = elg + erg
            e = jnp.where(x >= 0, x, x * 0.2)
            ee_v[pl.ds(o, LANES)] = jnp.exp(e - mvec)
        return 0
    lax.fori_loop(0, CAP1 // 64, step, 0)
    pltpu.sync_copy(ee_v, ee_hbm.at[wid])


def _make_sc_attention():
    mesh = plsc.VectorSubcoreMesh(core_axis_name="c", subcore_axis_name="s")
    return functools.partial(
        pl.kernel,
        out_type=[jax.ShapeDtypeStruct((NW, CAP1), jnp.float32)],
        mesh=mesh,
        compiler_params=_SC_PARAMS,
        scratch_types=[
            pltpu.VMEM((N_PAD,), jnp.float32),
            pltpu.VMEM((N_PAD,), jnp.float32),
            pltpu.VMEM((CAP1,), jnp.int32),
            pltpu.VMEM((CAP1,), jnp.int32),
            pltpu.VMEM((LANES,), jnp.float32),
            pltpu.VMEM((CAP1,), jnp.float32),
        ],
    )(_sc_attention)


def _zero_accumulator(u_sh, zero_v, sid, fext, zsem):
    zrow = jnp.zeros((LANES,), jnp.float32)

    def zb_row(zi, _):
        def zcol(qi, _):
            zero_v[zi, pl.ds(qi * LANES, LANES)] = zrow
            return 0
        lax.fori_loop(0, fext // LANES, zcol, 0)
        return 0
    lax.fori_loop(0, ZROWS, zb_row, 0)

    nz = RPT // ZROWS
    def zdma(ji, _):
        pltpu.async_copy(zero_v, u_sh.at[pl.ds(sid * RPT + ji * ZROWS, ZROWS)],
                         zsem)
        return 0
    lax.fori_loop(0, nz, zdma, 0)
    return nz


def _zero_drain(u_sh, zero_v, sid, nz, zsem):
    def zdrain(ji, _):
        pltpu.make_async_copy(
            zero_v, u_sh.at[pl.ds(sid * RPT, ZROWS)], zsem).wait()
        return 0
    lax.fori_loop(0, nz, zdrain, 0)


def _make_sc_scatter1():
    mesh = plsc.VectorSubcoreMesh(core_axis_name="c", subcore_axis_name="s")
    fext, k = F1E, K1

    def body(hx_hbm, ee_hbm, srcp_hbm, dstp_hbm, out_hbm,
             src_v, dst_v, ee_v, rows0_v, rows1_v, zero_v, u_sh,
             gsem0, gsem1, ssem0, ssem1, zsem):
        cid = lax.axis_index("c")
        sid = lax.axis_index("s")
        wid = sid * NC + cid
        cpt = jnp.where(cid == SLOW_CID, S1, F1)
        rows_bufs = (rows0_v, rows1_v)
        gsems = (gsem0, gsem1)
        ssems = (ssem0, ssem1)

        nz = _zero_accumulator(u_sh, zero_v, sid, fext, zsem)
        pltpu.sync_copy(srcp_hbm.at[wid], src_v)
        pltpu.sync_copy(dstp_hbm.at[wid], dst_v)
        pltpu.sync_copy(ee_hbm.at[wid], ee_v)
        _zero_drain(u_sh, zero_v, sid, nz, zsem)
        plsc.subcore_barrier()

        def gissue(ci, b):
            pltpu.async_copy(hx_hbm.at[src_v.at[ci]], rows_bufs[b], gsems[b])

        def gwait(b):
            pltpu.make_async_copy(hx_hbm.at[src_v.at[0]], rows_bufs[b],
                                  gsems[b]).wait()

        def sissue(ci, b):
            pltpu.async_copy(rows_bufs[b], u_sh.at[dst_v.at[ci]], ssems[b],
                             add=True)

        def swait(b):
            pltpu.make_async_copy(rows_bufs[b], u_sh.at[dst_v.at[0]],
                                  ssems[b]).wait()

        gissue(0, 0)

        def outer(g, _):
            for b in range(NBUF):
                c = g * NBUF + b
                buf = rows_bufs[b]

                @pl.when(c >= 1)
                def _():
                    swait(b ^ 1)

                @pl.when(c + 1 < cpt)
                def _():
                    gissue(c + 1, b ^ 1)
                gwait(b)

                def scale4(rg, _):
                    base = c * k + rg * 4
                    for rr in range(4):
                        ri = rg * 4 + rr
                        ee = plsc.load_gather(
                            ee_v, [jnp.full((LANES,), base + rr, jnp.int32)])
                        vals = [buf[ri, pl.ds(q * LANES, LANES)]
                                for q in range(fext // LANES)]
                        vals = [v * ee for v in vals]
                        for q in range(fext // LANES):
                            buf[ri, pl.ds(q * LANES, LANES)] = vals[q]
                    return 0
                lax.fori_loop(0, k // 4, scale4, 0)
                sissue(c, b)
            return 0
        lax.fori_loop(0, cpt // NBUF, outer, 0)
        swait(1)

        plsc.subcore_barrier()
        pltpu.sync_copy(u_sh.at[pl.ds(sid * RPT, RPT)],
                        out_hbm.at[cid, pl.ds(sid * RPT, RPT)])

    return functools.partial(
        pl.kernel,
        out_type=[jax.ShapeDtypeStruct((NC, N_PAD, fext), jnp.float32)],
        mesh=mesh,
        compiler_params=_SC_PARAMS,
        scratch_types=[
            pltpu.VMEM((F1, k), jnp.int32),
            pltpu.VMEM((F1, k), jnp.int32),
            pltpu.VMEM((CAP1,), jnp.float32),
            pltpu.VMEM((k, fext), jnp.float32),
            pltpu.VMEM((k, fext), jnp.float32),
            pltpu.VMEM((ZROWS, fext), jnp.float32),
            pltpu.VMEM_SHARED((N_UPAD, fext), jnp.float32),
            pltpu.SemaphoreType.DMA,
            pltpu.SemaphoreType.DMA,
            pltpu.SemaphoreType.DMA,
            pltpu.SemaphoreType.DMA,
            pltpu.SemaphoreType.DMA,
        ],
    )(body)


def _make_sc_edge2():
    mesh = plsc.VectorSubcoreMesh(core_axis_name="c", subcore_axis_name="s")
    fext, k = F2E, K2

    def body(hx_hbm, eler_hbm, mvec_hbm, srcp_hbm, dstp_hbm, out_hbm,
             el_v, er_v, src_v, dst_v, m_v, ee_v, rows0_v, rows1_v, zero_v,
             u_sh, gsem0, gsem1, ssem0, ssem1, zsem):
        cid = lax.axis_index("c")
        sid = lax.axis_index("s")
        wid = sid * NC + cid
        cpt = jnp.where(cid == SLOW_CID, S2, F2)
        rows_bufs = (rows0_v, rows1_v)
        gsems = (gsem0, gsem1)
        ssems = (ssem0, ssem1)

        nz = _zero_accumulator(u_sh, zero_v, sid, fext, zsem)
        pltpu.sync_copy(eler_hbm.at[0], el_v)
        pltpu.sync_copy(eler_hbm.at[1], er_v)
        pltpu.sync_copy(mvec_hbm, m_v)
        pltpu.sync_copy(srcp_hbm.at[wid], src_v)
        pltpu.sync_copy(dstp_hbm.at[wid], dst_v)
        _zero_drain(u_sh, zero_v, sid, nz, zsem)
        plsc.subcore_barrier()
        mvec = m_v[...]

        def gissue(ci, b):
            pltpu.async_copy(hx_hbm.at[src_v.at[ci]], rows_bufs[b], gsems[b])

        def gwait(b):
            pltpu.make_async_copy(hx_hbm.at[src_v.at[0]], rows_bufs[b],
                                  gsems[b]).wait()

        def sissue(ci, b):
            pltpu.async_copy(rows_bufs[b], u_sh.at[dst_v.at[ci]], ssems[b],
                             add=True)

        def swait(b):
            pltpu.make_async_copy(rows_bufs[b], u_sh.at[dst_v.at[0]],
                                  ssems[b]).wait()

        gissue(0, 0)

        def outer(g, _):
            for b in range(NBUF):
                c = g * NBUF + b
                buf = rows_bufs[b]

                @pl.when(c >= 1)
                def _():
                    swait(b ^ 1)

                @pl.when(c + 1 < cpt)
                def _():
                    gissue(c + 1, b ^ 1)

                for j in range(k // LANES):
                    sv = src_v[c, pl.ds(j * LANES, LANES)]
                    dv = dst_v[c, pl.ds(j * LANES, LANES)]
                    elg = plsc.load_gather(el_v, [sv])
                    erg = plsc.load_gather(er_v, [dv])
                    x = elg + erg
                    e = jnp.where(x >= 0, x, x * 0.2)
                    ee_v[pl.ds(j * LANES, LANES)] = jnp.exp(e - mvec)
                gwait(b)

                def scale4(rg, _):
                    for rr in range(4):
                        ri = rg * 4 + rr
                        ee = plsc.load_gather(
                            ee_v, [jnp.full((LANES,), ri, jnp.int32)])
                        vals = [buf[ri, pl.ds(q * LANES, LANES)]
                                for q in range(fext // LANES)]
                        vals = [v * ee for v in vals]
                        for q in range(fext // LANES):
                            buf[ri, pl.ds(q * LANES, LANES)] = vals[q]
                    return 0
                lax.fori_loop(0, k // 4, scale4, 0)
                sissue(c, b)
            return 0
        lax.fori_loop(0, cpt // NBUF, outer, 0)
        swait(1)

        plsc.subcore_barrier()
        pltpu.sync_copy(u_sh.at[pl.ds(sid * RPT, RPT)],
                        out_hbm.at[cid, pl.ds(sid * RPT, RPT)])

    return functools.partial(
        pl.kernel,
        out_type=[jax.ShapeDtypeStruct((NC, N_PAD, fext), jnp.float32)],
        mesh=mesh,
        compiler_params=_SC_PARAMS,
        scratch_types=[
            pltpu.VMEM((N_PAD,), jnp.float32),
            pltpu.VMEM((N_PAD,), jnp.float32),
            pltpu.VMEM((F2, k), jnp.int32),
            pltpu.VMEM((F2, k), jnp.int32),
            pltpu.VMEM((LANES,), jnp.float32),
            pltpu.VMEM((k + LANES,), jnp.float32),
            pltpu.VMEM((k, fext), jnp.float32),
            pltpu.VMEM((k, fext), jnp.float32),
            pltpu.VMEM((ZROWS, fext), jnp.float32),
            pltpu.VMEM_SHARED((N_UPAD, fext), jnp.float32),
            pltpu.SemaphoreType.DMA,
            pltpu.SemaphoreType.DMA,
            pltpu.SemaphoreType.DMA,
            pltpu.SemaphoreType.DMA,
            pltpu.SemaphoreType.DMA,
        ],
    )(body)


_sc_att_1 = _make_sc_attention()
_sc_scatter_1 = _make_sc_scatter1()
_sc_edge_2 = _make_sc_edge2()


def kernel(edge_index, feat, unsplice, splice, alpha0, beta0, gamma0, dt,
           W1, b1, al1, ar1, W2, b2, al2, ar2, Wl, bl):
    f32 = jnp.float32
    src = edge_index[0]
    dst = edge_index[1]
    pad_e = E_PAD - E
    src_p = jnp.concatenate([src, jnp.zeros((pad_e,), jnp.int32)])
    dst_p = jnp.concatenate([dst, jnp.full((pad_e,), N, jnp.int32)])

    def _distribute(slow_k, fast_k, cap):
        cids = jnp.arange(NW, dtype=jnp.int32) % NC
        counts = jnp.where(cids == SLOW_CID, slow_k, fast_k)
        off = jnp.cumsum(counts) - counts
        j = jnp.arange(cap, dtype=jnp.int32)
        idx = off[:, None] + jnp.minimum(j[None, :], counts[:, None] - 1)
        valid = j[None, :] < counts[:, None]
        sw = jnp.where(valid, src_p[idx], 0)
        dw = jnp.where(valid, dst_p[idx], N)
        return sw, dw

    srcw1, dstw1 = _distribute(S1 * K1, F1 * K1, CAP1)
    srcp1 = srcw1.reshape(NW, F1, K1)
    dstp1 = dstw1.reshape(NW, F1, K1)
    srcw2, dstw2 = _distribute(S2 * K2, F2 * K2, CAP2)
    srcp2 = srcw2.reshape(NW, F2, K2)
    dstp2 = dstw2.reshape(NW, F2, K2)

    feat_p = jnp.zeros((N_PAD, IN_FEATS), f32).at[:N].set(feat)
    w1p = jnp.zeros((F1E, IN_FEATS), f32).at[:H1].set(W1)
    alr1 = jnp.zeros((2, F1E), f32).at[0, :H1].set(al1[0]).at[1, :H1].set(ar1[0])
    w2p = jnp.zeros((F2E, H1), f32).at[:H2].set(W2)
    alr2 = jnp.zeros((2, F2E), f32).at[0, :H2].set(al2[0]).at[1, :H2].set(ar2[0])
    wlp = jnp.zeros((8, H2), f32).at[:3].set(Wl)
    blp = jnp.zeros((8, 1), f32).at[:3, 0].set(bl)
    usp = jnp.zeros((1, N_PAD), f32).at[0, :N].set(unsplice)
    spp = jnp.zeros((1, N_PAD), f32).at[0, :N].set(splice)
    scal = jnp.stack([alpha0[0], beta0[0], gamma0[0], dt[0]]).reshape(1, 4)

    hx1, eler1, mm1 = _tc_pre(feat_p, w1p, alr1)
    mvec1 = jnp.full((LANES,), mm1[0, 0] + mm1[1, 0], f32)
    (ee1,) = _sc_att_1(eler1, mvec1, srcw1, dstw1)
    (u1,) = _sc_scatter_1(hx1, ee1, srcp1, dstp1)

    hx2, eler2, mm2 = _tc_mid(u1[0], u1[1], b1.reshape(1, H1), w2p, alr2)
    mvec2 = jnp.full((LANES,), mm2[0, 0] + mm2[1, 0], f32)
    (u2,) = _sc_edge_2(hx2, eler2, mvec2, srcp2, dstp2)

    (out8,) = _tc_final(u2[0], u2[1], b2.reshape(1, H2), wlp, blp, usp, spp, scal)

    return (out8[0, :N], out8[1, :N], out8[2, :N], out8[3, :N], out8[4, :N])


# speedup vs baseline: 1.6679x; 1.4516x over previous
import functools

import numpy as np

import jax
import jax.numpy as jnp
from jax import lax
from jax.experimental import pallas as pl
from jax.experimental.pallas import tpu as pltpu
from jax.experimental.pallas import tpu_sc as plsc

N = 10000
E = 160000
IN_FEATS = 128
H1 = 128
H2 = 64

NC = 2
NS = 16
NW = NC * NS
LANES = 16

N_PAD = 10240
N_UPAD = 10048
R = 512
NBUF = 2
K1, CPT1 = 48, 108
K2, CPT2 = 128, 40
EPT1 = K1 * CPT1
EPT2 = K2 * CPT2
E_PAD1 = EPT1 * NW
E_PAD2 = EPT2 * NW

F1E = 144
F2E = 80
F1B = 160
F2B = 96
ZROWS = 4
RPT = N_UPAD // NS


def _unpack_order(fbf):
    c_of_p = np.zeros(fbf, dtype=np.int64)
    for p in range(fbf):
        j, t = divmod(p, 32)
        c_of_p[p] = 32 * j + (t // 2 if t % 2 == 0 else 16 + (t - 1) // 2)
    return c_of_p


_C_OF_P1 = _unpack_order(F1B)
_C_OF_P2 = _unpack_order(F2B)


def _elu(x):
    return jnp.where(x > 0, x, jnp.exp(jnp.minimum(x, 0.0)) - 1.0)


def _tc_pre_body(f_ref, w_ref, alr_ref, hx_ref, eler_ref, mm_ref):
    i = pl.program_id(0)
    ones_lane = hx_ref.shape[1] - 32
    h = lax.dot_general(f_ref[...], w_ref[...], (((1,), (1,)), ((), ())),
                        preferred_element_type=jnp.float32)
    lane = lax.broadcasted_iota(jnp.int32, h.shape, 1)
    h = h + jnp.where(lane == ones_lane, 1.0, 0.0)
    hx_ref[...] = h.astype(jnp.bfloat16)
    eler = lax.dot_general(alr_ref[...], h, (((1,), (1,)), ((), ())),
                           preferred_element_type=jnp.float32)
    eler_ref[...] = eler
    mblk = jnp.max(eler, axis=1, keepdims=True)

    @pl.when(i == 0)
    def _():
        mm_ref[...] = mblk

    @pl.when(i > 0)
    def _():
        mm_ref[...] = jnp.maximum(mm_ref[...], mblk)


def _tc_pre(feat_p, w1p, alr1):
    grid = (N_PAD // R,)
    return pl.pallas_call(
        _tc_pre_body,
        grid=grid,
        in_specs=[
            pl.BlockSpec((R, IN_FEATS), lambda i: (i, 0)),
            pl.BlockSpec((F1B, IN_FEATS), lambda i: (0, 0)),
            pl.BlockSpec((2, F1B), lambda i: (0, 0)),
        ],
        out_specs=[
            pl.BlockSpec((R, F1B), lambda i: (i, 0)),
            pl.BlockSpec((2, R), lambda i: (0, i)),
            pl.BlockSpec((2, 1), lambda i: (0, 0)),
        ],
        out_shape=[
            jax.ShapeDtypeStruct((N_PAD, F1B), jnp.bfloat16),
            jax.ShapeDtypeStruct((2, N_PAD), jnp.float32),
            jax.ShapeDtypeStruct((2, 1), jnp.float32),
        ],
    )(feat_p, w1p, alr1)


def _tc_mid_body(ua_ref, ub_ref, b1_ref, w2_ref, alr_ref, hx_ref, eler_ref,
                 mm_ref):
    i = pl.program_id(0)
    ones_lane = hx_ref.shape[1] - 32
    u = ua_ref[...] + ub_ref[...]
    denom = jnp.maximum(u[:, H1:H1 + 1], 1e-9)
    rst = u[:, :H1] / denom + b1_ref[...]
    x = _elu(_elu(rst))
    h = lax.dot_general(x, w2_ref[...], (((1,), (1,)), ((), ())),
                        preferred_element_type=jnp.float32)
    lane = lax.broadcasted_iota(jnp.int32, h.shape, 1)
    h = h + jnp.where(lane == ones_lane, 1.0, 0.0)
    hx_ref[...] = h.astype(jnp.bfloat16)
    eler = lax.dot_general(alr_ref[...], h, (((1,), (1,)), ((), ())),
                           preferred_element_type=jnp.float32)
    row = lax.broadcasted_iota(jnp.int32, eler.shape, 1) + i * R
    eler = jnp.where(row < N_UPAD, eler, -1e30)
    eler_ref[...] = eler
    mblk = jnp.max(eler, axis=1, keepdims=True)

    @pl.when(i == 0)
    def _():
        mm_ref[...] = mblk

    @pl.when(i > 0)
    def _():
        mm_ref[...] = jnp.maximum(mm_ref[...], mblk)


def _tc_mid(ua, ub, b1, w2p, alr2):
    grid = (N_PAD // R,)
    return pl.pallas_call(
        _tc_mid_body,
        grid=grid,
        in_specs=[
            pl.BlockSpec((R, F1E), lambda i: (i, 0)),
            pl.BlockSpec((R, F1E), lambda i: (i, 0)),
            pl.BlockSpec((1, H1), lambda i: (0, 0)),
            pl.BlockSpec((F2B, H1), lambda i: (0, 0)),
            pl.BlockSpec((2, F2B), lambda i: (0, 0)),
        ],
        out_specs=[
            pl.BlockSpec((R, F2B), lambda i: (i, 0)),
            pl.BlockSpec((2, R), lambda i: (0, i)),
            pl.BlockSpec((2, 1), lambda i: (0, 0)),
        ],
        out_shape=[
            jax.ShapeDtypeStruct((N_PAD, F2B), jnp.bfloat16),
            jax.ShapeDtypeStruct((2, N_PAD), jnp.float32),
            jax.ShapeDtypeStruct((2, 1), jnp.float32),
        ],
    )(ua, ub, b1, w2p, alr2)


def _tc_final_body(ua_ref, ub_ref, b2_ref, wl_ref, bl_ref, us_ref, sp_ref,
                   scal_ref, out_ref):
    u = ua_ref[...] + ub_ref[...]
    denom = jnp.maximum(u[:, H2:H2 + 1], 1e-9)
    x = _elu(u[:, :H2] / denom + b2_ref[...])
    zt = lax.dot_general(wl_ref[...], x, (((1,), (1,)), ((), ())),
                         preferred_element_type=jnp.float32)
    zt = zt + bl_ref[...]
    sig = 1.0 / (1.0 + jnp.exp(-zt))
    alpha0 = scal_ref[0:1, 0:1]
    beta0 = scal_ref[0:1, 1:2]
    gamma0 = scal_ref[0:1, 2:3]
    dt = scal_ref[0:1, 3:4]
    beta = sig[0:1, :] * beta0
    gamma = sig[1:2, :] * gamma0
    alphas = sig[2:3, :] * alpha0
    us = us_ref[...]
    sp = sp_ref[...]
    up_out = us + (alphas - beta * us) * dt
    sp_out = sp + (beta * us - gamma * sp) * dt
    zero3 = jnp.zeros((3, up_out.shape[1]), jnp.float32)
    out_ref[...] = jnp.concatenate([up_out, sp_out, alphas, beta, gamma, zero3], 0)


def _tc_final(ua, ub, b2, wlp, blp, us, sp, scal):
    grid = (N_PAD // R,)
    return pl.pallas_call(
        _tc_final_body,
        grid=grid,
        in_specs=[
            pl.BlockSpec((R, F2E), lambda i: (i, 0)),
            pl.BlockSpec((R, F2E), lambda i: (i, 0)),
            pl.BlockSpec((1, H2), lambda i: (0, 0)),
            pl.BlockSpec((8, H2), lambda i: (0, 0)),
            pl.BlockSpec((8, 1), lambda i: (0, 0)),
            pl.BlockSpec((1, R), lambda i: (0, i)),
            pl.BlockSpec((1, R), lambda i: (0, i)),
            pl.BlockSpec((1, 4), lambda i: (0, 0)),
        ],
        out_specs=[pl.BlockSpec((8, R), lambda i: (0, i))],
        out_shape=[jax.ShapeDtypeStruct((8, N_PAD), jnp.float32)],
    )(ua, ub, b2, wlp, blp, us, sp, scal)


_SC_PARAMS = pltpu.CompilerParams(
    needs_layout_passes=False, use_tc_tiling_on_sc=False)


def _sc_attention(eler_hbm, mvec_hbm, srcp_hbm, dstp_hbm, ee_hbm,
                  el_v, er_v, src_v, dst_v, m_v, ee_v):
    cid = lax.axis_index("c")
    sid = lax.axis_index("s")
    wid = sid * NC + cid
    pltpu.sync_copy(eler_hbm.at[0], el_v)
    pltpu.sync_copy(eler_hbm.at[1], er_v)
    pltpu.sync_copy(mvec_hbm, m_v)
    pltpu.sync_copy(srcp_hbm.at[wid], src_v)
    pltpu.sync_copy(dstp_hbm.at[wid], dst_v)
    mvec = m_v[...]

    def step(t, _):
        for j in range(4):
            o = t * 64 + j * LANES
            sv = src_v[pl.ds(o, LANES)]
            dv = dst_v[pl.ds(o, LANES)]
            elg = plsc.load_gather(el_v, [sv])
            erg = plsc.load_gather(er_v, [dv])
            x = elg + erg
            e = jnp.where(x >= 0, x, x * 0.2)
            ee_v[pl.ds(o, LANES)] = jnp.exp(e - mvec)
        return 0
    lax.fori_loop(0, EPT1 // 64, step, 0)
    pltpu.sync_copy(ee_v, ee_hbm.at[wid])


def _make_sc_attention():
    mesh = plsc.VectorSubcoreMesh(core_axis_name="c", subcore_axis_name="s")
    return functools.partial(
        pl.kernel,
        out_type=[jax.ShapeDtypeStruct((NW, EPT1), jnp.float32)],
        mesh=mesh,
        compiler_params=_SC_PARAMS,
        scratch_types=[
            pltpu.VMEM((N_PAD,), jnp.float32),
            pltpu.VMEM((N_PAD,), jnp.float32),
            pltpu.VMEM((EPT1,), jnp.int32),
            pltpu.VMEM((EPT1,), jnp.int32),
            pltpu.VMEM((LANES,), jnp.float32),
            pltpu.VMEM((EPT1,), jnp.float32),
        ],
    )(_sc_attention)


def _zero_accumulator(u_sh, zero_v, sid, fext, zsem):
    zrow = jnp.zeros((LANES,), jnp.float32)

    def zb_row(zi, _):
        def zcol(qi, _):
            zero_v[zi, pl.ds(qi * LANES, LANES)] = zrow
            return 0
        lax.fori_loop(0, fext // LANES, zcol, 0)
        return 0
    lax.fori_loop(0, ZROWS, zb_row, 0)

    nz = RPT // ZROWS
    def zdma(ji, _):
        pltpu.async_copy(zero_v, u_sh.at[pl.ds(sid * RPT + ji * ZROWS, ZROWS)],
                         zsem)
        return 0
    lax.fori_loop(0, nz, zdma, 0)
    return nz


def _zero_drain(u_sh, zero_v, sid, nz, zsem):
    def zdrain(ji, _):
        pltpu.make_async_copy(
            zero_v, u_sh.at[pl.ds(sid * RPT, ZROWS)], zsem).wait()
        return 0
    lax.fori_loop(0, nz, zdrain, 0)


def _unpack_scale_store(bfbuf, fbuf, ri, ee, fbf, fext):
    mask_hi = jnp.full((LANES,), -65536, jnp.int32)
    for blk in range(fbf // 32):
        v = bfbuf[ri, pl.ds(blk * 32, 32)]
        w = plsc.bitcast(v, jnp.int32)
        lo = plsc.bitcast(lax.shift_left(w, 16), jnp.float32)
        fbuf[ri, pl.ds(blk * 32, LANES)] = lo * ee
        if blk * 32 + LANES < fext:
            hi = plsc.bitcast(jnp.bitwise_and(w, mask_hi), jnp.float32)
            fbuf[ri, pl.ds(blk * 32 + LANES, LANES)] = hi * ee


def _make_sc_scatter1():
    mesh = plsc.VectorSubcoreMesh(core_axis_name="c", subcore_axis_name="s")
    fext, fbf, k, cpt = F1E, F1B, K1, CPT1

    def body(hx_hbm, ee_hbm, srcp_hbm, dstp_hbm, out_hbm,
             src_v, dst_v, ee_v, bf0_v, bf1_v, f0_v, f1_v, zero_v, u_sh,
             gsem0, gsem1, ssem0, ssem1, zsem):
        cid = lax.axis_index("c")
        sid = lax.axis_index("s")
        wid = sid * NC + cid
        bf_bufs = (bf0_v, bf1_v)
        f_bufs = (f0_v, f1_v)
        gsems = (gsem0, gsem1)
        ssems = (ssem0, ssem1)

        nz = _zero_accumulator(u_sh, zero_v, sid, fext, zsem)
        pltpu.sync_copy(srcp_hbm.at[wid], src_v)
        pltpu.sync_copy(dstp_hbm.at[wid], dst_v)
        pltpu.sync_copy(ee_hbm.at[wid], ee_v)
        _zero_drain(u_sh, zero_v, sid, nz, zsem)
        plsc.subcore_barrier()

        def gissue(ci, b):
            pltpu.async_copy(hx_hbm.at[src_v.at[ci]], bf_bufs[b], gsems[b])

        def gwait(b):
            pltpu.make_async_copy(hx_hbm.at[src_v.at[0]], bf_bufs[b],
                                  gsems[b]).wait()

        def sissue(ci, b):
            pltpu.async_copy(f_bufs[b], u_sh.at[dst_v.at[ci]], ssems[b],
                             add=True)

        def swait(b):
            pltpu.make_async_copy(f_bufs[b], u_sh.at[dst_v.at[0]],
                                  ssems[b]).wait()

        gissue(0, 0)

        def outer(g, _):
            for b in range(NBUF):
                c = g * NBUF + b

                @pl.when(c + 1 < cpt)
                def _():
                    gissue(c + 1, b ^ 1)
                gwait(b)

                @pl.when(c >= 2)
                def _():
                    swait(b)

                def scale4(rg, _):
                    base = c * k + rg * 4
                    for rr in range(4):
                        ri = rg * 4 + rr
                        ee = plsc.load_gather(
                            ee_v, [jnp.full((LANES,), base + rr, jnp.int32)])
                        _unpack_scale_store(bf_bufs[b], f_bufs[b], ri, ee,
                                            fbf, fext)
                    return 0
                lax.fori_loop(0, k // 4, scale4, 0)
                sissue(c, b)
            return 0
        lax.fori_loop(0, cpt // NBUF, outer, 0)
        swait(0)
        swait(1)

        plsc.subcore_barrier()
        pltpu.sync_copy(u_sh.at[pl.ds(sid * RPT, RPT)],
                        out_hbm.at[cid, pl.ds(sid * RPT, RPT)])

    return functools.partial(
        pl.kernel,
        out_type=[jax.ShapeDtypeStruct((NC, N_PAD, fext), jnp.float32)],
        mesh=mesh,
        compiler_params=_SC_PARAMS,
        scratch_types=[
            pltpu.VMEM((cpt, k), jnp.int32),
            pltpu.VMEM((cpt, k), jnp.int32),
            pltpu.VMEM((EPT1,), jnp.float32),
            pltpu.VMEM((k, fbf), jnp.bfloat16),
            pltpu.VMEM((k, fbf), jnp.bfloat16),
            pltpu.VMEM((k, fext), jnp.float32),
            pltpu.VMEM((k, fext), jnp.float32),
            pltpu.VMEM((ZROWS, fext), jnp.float32),
            pltpu.VMEM_SHARED((N_UPAD, fext), jnp.float32),
            pltpu.SemaphoreType.DMA,
            pltpu.SemaphoreType.DMA,
            pltpu.SemaphoreType.DMA,
            pltpu.SemaphoreType.DMA,
            pltpu.SemaphoreType.DMA,
        ],
    )(body)


def _make_sc_edge2():
    mesh = plsc.VectorSubcoreMesh(core_axis_name="c", subcore_axis_name="s")
    fext, fbf, k, cpt = F2E, F2B, K2, CPT2

    def body(hx_hbm, eler_hbm, mvec_hbm, srcp_hbm, dstp_hbm, out_hbm,
             el_v, er_v, src_v, dst_v, m_v, ee_v, bf0_v, bf1_v, f0_v, f1_v,
             zero_v, u_sh, gsem0, gsem1, ssem0, ssem1, zsem):
        cid = lax.axis_index("c")
        sid = lax.axis_index("s")
        wid = sid * NC + cid
        bf_bufs = (bf0_v, bf1_v)
        f_bufs = (f0_v, f1_v)
        gsems = (gsem0, gsem1)
        ssems = (ssem0, ssem1)

        nz = _zero_accumulator(u_sh, zero_v, sid, fext, zsem)
        pltpu.sync_copy(eler_hbm.at[0], el_v)
        pltpu.sync_copy(eler_hbm.at[1], er_v)
        pltpu.sync_copy(mvec_hbm, m_v)
        pltpu.sync_copy(srcp_hbm.at[wid], src_v)
        pltpu.sync_copy(dstp_hbm.at[wid], dst_v)
        _zero_drain(u_sh, zero_v, sid, nz, zsem)
        plsc.subcore_barrier()
        mvec = m_v[...]

        def gissue(ci, b):
            pltpu.async_copy(hx_hbm.at[src_v.at[ci]], bf_bufs[b], gsems[b])

        def gwait(b):
            pltpu.make_async_copy(hx_hbm.at[src_v.at[0]], bf_bufs[b],
                                  gsems[b]).wait()

        def sissue(ci, b):
            pltpu.async_copy(f_bufs[b], u_sh.at[dst_v.at[ci]], ssems[b],
                             add=True)

        def swait(b):
            pltpu.make_async_copy(f_bufs[b], u_sh.at[dst_v.at[0]],
                                  ssems[b]).wait()

        gissue(0, 0)

        def outer(g, _):
            for b in range(NBUF):
                c = g * NBUF + b

                @pl.when(c + 1 < cpt)
                def _():
                    gissue(c + 1, b ^ 1)

                for j in range(k // LANES):
                    sv = src_v[c, pl.ds(j * LANES, LANES)]
                    dv = dst_v[c, pl.ds(j * LANES, LANES)]
                    elg = plsc.load_gather(el_v, [sv])
                    erg = plsc.load_gather(er_v, [dv])
                    x = elg + erg
                    e = jnp.where(x >= 0, x, x * 0.2)
                    ee_v[pl.ds(j * LANES, LANES)] = jnp.exp(e - mvec)
                gwait(b)

                @pl.when(c >= 2)
                def _():
                    swait(b)

                def scale4(rg, _):
                    for rr in range(4):
                        ri = rg * 4 + rr
                        ee = plsc.load_gather(
                            ee_v, [jnp.full((LANES,), ri, jnp.int32)])
                        _unpack_scale_store(bf_bufs[b], f_bufs[b], ri, ee,
                                            fbf, fext)
                    return 0
                lax.fori_loop(0, k // 4, scale4, 0)
                sissue(c, b)
            return 0
        lax.fori_loop(0, cpt // NBUF, outer, 0)
        swait(0)
        swait(1)

        plsc.subcore_barrier()
        pltpu.sync_copy(u_sh.at[pl.ds(sid * RPT, RPT)],
                        out_hbm.at[cid, pl.ds(sid * RPT, RPT)])

    return functools.partial(
        pl.kernel,
        out_type=[jax.ShapeDtypeStruct((NC, N_PAD, fext), jnp.float32)],
        mesh=mesh,
        compiler_params=_SC_PARAMS,
        scratch_types=[
            pltpu.VMEM((N_PAD,), jnp.float32),
            pltpu.VMEM((N_PAD,), jnp.float32),
            pltpu.VMEM((cpt, k), jnp.int32),
            pltpu.VMEM((cpt, k), jnp.int32),
            pltpu.VMEM((LANES,), jnp.float32),
            pltpu.VMEM((k,), jnp.float32),
            pltpu.VMEM((k, fbf), jnp.bfloat16),
            pltpu.VMEM((k, fbf), jnp.bfloat16),
            pltpu.VMEM((k, fext), jnp.float32),
            pltpu.VMEM((k, fext), jnp.float32),
            pltpu.VMEM((ZROWS, fext), jnp.float32),
            pltpu.VMEM_SHARED((N_UPAD, fext), jnp.float32),
            pltpu.SemaphoreType.DMA,
            pltpu.SemaphoreType.DMA,
            pltpu.SemaphoreType.DMA,
            pltpu.SemaphoreType.DMA,
            pltpu.SemaphoreType.DMA,
        ],
    )(body)


_sc_att_1 = _make_sc_attention()
_sc_scatter_1 = _make_sc_scatter1()
_sc_edge_2 = _make_sc_edge2()


def _permute_rows(w_ext, al_ext, c_of_p, fbf):
    ncols = w_ext.shape[0]
    sel = np.nonzero(c_of_p < ncols)[0]
    csel = c_of_p[sel]
    wp = jnp.zeros((fbf, w_ext.shape[1]), jnp.float32).at[sel].set(w_ext[csel])
    alp = jnp.zeros((2, fbf), jnp.float32).at[:, sel].set(al_ext[:, csel])
    return wp, alp


def kernel(edge_index, feat, unsplice, splice, alpha0, beta0, gamma0, dt,
           W1, b1, al1, ar1, W2, b2, al2, ar2, Wl, bl):
    f32 = jnp.float32
    src = edge_index[0]
    dst = edge_index[1]
    src_p1 = jnp.concatenate([src, jnp.zeros((E_PAD1 - E,), jnp.int32)])
    dst_p1 = jnp.concatenate([dst, jnp.full((E_PAD1 - E,), N, jnp.int32)])
    src_p2 = jnp.concatenate([src, jnp.zeros((E_PAD2 - E,), jnp.int32)])
    dst_p2 = jnp.concatenate([dst, jnp.full((E_PAD2 - E,), N, jnp.int32)])
    srcp1 = src_p1.reshape(NW, CPT1, K1)
    dstp1 = dst_p1.reshape(NW, CPT1, K1)
    srcp1f = src_p1.reshape(NW, EPT1)
    dstp1f = dst_p1.reshape(NW, EPT1)
    srcp2 = src_p2.reshape(NW, CPT2, K2)
    dstp2 = dst_p2.reshape(NW, CPT2, K2)

    feat_p = jnp.zeros((N_PAD, IN_FEATS), f32).at[:N].set(feat)
    alr1e = jnp.concatenate([al1, ar1], 0)
    w1p, alr1 = _permute_rows(W1, alr1e, _C_OF_P1, F1B)
    alr2e = jnp.concatenate([al2, ar2], 0)
    w2p, alr2 = _permute_rows(W2, alr2e, _C_OF_P2, F2B)
    wlp = jnp.zeros((8, H2), f32).at[:3].set(Wl)
    blp = jnp.zeros((8, 1), f32).at[:3, 0].set(bl)
    usp = jnp.zeros((1, N_PAD), f32).at[0, :N].set(unsplice)
    spp = jnp.zeros((1, N_PAD), f32).at[0, :N].set(splice)
    scal = jnp.stack([alpha0[0], beta0[0], gamma0[0], dt[0]]).reshape(1, 4)

    hx1, eler1, mm1 = _tc_pre(feat_p, w1p, alr1)
    mvec1 = jnp.full((LANES,), mm1[0, 0] + mm1[1, 0], f32)
    (ee1,) = _sc_att_1(eler1, mvec1, srcp1f, dstp1f)
    (u1,) = _sc_scatter_1(hx1, ee1, srcp1, dstp1)

    hx2, eler2, mm2 = _tc_mid(u1[0], u1[1], b1.reshape(1, H1), w2p, alr2)
    mvec2 = jnp.full((LANES,), mm2[0, 0] + mm2[1, 0], f32)
    (u2,) = _sc_edge_2(hx2, eler2, mvec2, srcp2, dstp2)

    (out8,) = _tc_final(u2[0], u2[1], b2.reshape(1, H2), wlp, blp, usp, spp, scal)

    return (out8[0, :N], out8[1, :N], out8[2, :N], out8[3, :N], out8[4, :N])


# speedup vs baseline: 2.0338x; 1.2193x over previous
import functools

import numpy as np

import jax
import jax.numpy as jnp
from jax import lax
from jax.experimental import pallas as pl
from jax.experimental.pallas import tpu as pltpu
from jax.experimental.pallas import tpu_sc as plsc

N = 10000
E = 160000
IN_FEATS = 128
H1 = 128
H2 = 64

NC = 2
NS = 16
NW = NC * NS
LANES = 16

N_PAD = 10240
N_UPAD = 10048
R = 512
NBUF = 2
K1, CPT1 = 128, 40
K2, CPT2 = 128, 40
EPT1 = K1 * CPT1
EPT2 = K2 * CPT2
E_PAD1 = EPT1 * NW
E_PAD2 = EPT2 * NW

F1B = 128
F2B = 64
DW = 16
ZROWS = 4
RPT = N_UPAD // NS


def _unpack_order(fbf):
    c_of_p = np.zeros(fbf, dtype=np.int64)
    for p in range(fbf):
        j, t = divmod(p, 32)
        c_of_p[p] = 32 * j + (t // 2 if t % 2 == 0 else 16 + (t - 1) // 2)
    return c_of_p


_C_OF_P1 = _unpack_order(F1B)
_C_OF_P2 = _unpack_order(F2B)


def _elu(x):
    return jnp.where(x > 0, x, jnp.exp(jnp.minimum(x, 0.0)) - 1.0)


def _tc_pre_body(f_ref, w_ref, alr_ref, hx_ref, eler_ref, mm_ref):
    i = pl.program_id(0)
    h = lax.dot_general(f_ref[...], w_ref[...], (((1,), (1,)), ((), ())),
                        preferred_element_type=jnp.float32)
    hx_ref[...] = h.astype(jnp.bfloat16)
    eler = lax.dot_general(alr_ref[...], h, (((1,), (1,)), ((), ())),
                           preferred_element_type=jnp.float32)
    eler_ref[...] = eler
    mblk = jnp.max(eler, axis=1, keepdims=True)

    @pl.when(i == 0)
    def _():
        mm_ref[...] = mblk

    @pl.when(i > 0)
    def _():
        mm_ref[...] = jnp.maximum(mm_ref[...], mblk)


def _tc_pre(feat_p, w1p, alr1):
    grid = (N_PAD // R,)
    return pl.pallas_call(
        _tc_pre_body,
        grid=grid,
        in_specs=[
            pl.BlockSpec((R, IN_FEATS), lambda i: (i, 0)),
            pl.BlockSpec((F1B, IN_FEATS), lambda i: (0, 0)),
            pl.BlockSpec((2, F1B), lambda i: (0, 0)),
        ],
        out_specs=[
            pl.BlockSpec((R, F1B), lambda i: (i, 0)),
            pl.BlockSpec((2, R), lambda i: (0, i)),
            pl.BlockSpec((2, 1), lambda i: (0, 0)),
        ],
        out_shape=[
            jax.ShapeDtypeStruct((N_PAD, F1B), jnp.bfloat16),
            jax.ShapeDtypeStruct((2, N_PAD), jnp.float32),
            jax.ShapeDtypeStruct((2, 1), jnp.float32),
        ],
    )(feat_p, w1p, alr1)


def _tc_mid_body(ufa_ref, ufb_ref, uda_ref, udb_ref, b1_ref, w2_ref, alr_ref,
                 hx_ref, eler_ref, mm_ref):
    i = pl.program_id(0)
    uf = (ufa_ref[...].astype(jnp.float32) + ufb_ref[...].astype(jnp.float32))
    den = uda_ref[:, 0:1] + udb_ref[:, 0:1]
    denom = jnp.maximum(den, 1e-9)
    rst = uf / denom + b1_ref[...]
    x = _elu(_elu(rst))
    h = lax.dot_general(x, w2_ref[...], (((1,), (1,)), ((), ())),
                        preferred_element_type=jnp.float32)
    hx_ref[...] = h.astype(jnp.bfloat16)
    eler = lax.dot_general(alr_ref[...], h, (((1,), (1,)), ((), ())),
                           preferred_element_type=jnp.float32)
    row = lax.broadcasted_iota(jnp.int32, eler.shape, 1) + i * R
    eler = jnp.where(row < N_UPAD, eler, -1e30)
    eler_ref[...] = eler
    mblk = jnp.max(eler, axis=1, keepdims=True)

    @pl.when(i == 0)
    def _():
        mm_ref[...] = mblk

    @pl.when(i > 0)
    def _():
        mm_ref[...] = jnp.maximum(mm_ref[...], mblk)


def _tc_mid(ufa, ufb, uda, udb, b1, w2p, alr2):
    grid = (N_PAD // R,)
    return pl.pallas_call(
        _tc_mid_body,
        grid=grid,
        in_specs=[
            pl.BlockSpec((R, F1B), lambda i: (i, 0)),
            pl.BlockSpec((R, F1B), lambda i: (i, 0)),
            pl.BlockSpec((R, DW), lambda i: (i, 0)),
            pl.BlockSpec((R, DW), lambda i: (i, 0)),
            pl.BlockSpec((1, H1), lambda i: (0, 0)),
            pl.BlockSpec((F2B, H1), lambda i: (0, 0)),
            pl.BlockSpec((2, F2B), lambda i: (0, 0)),
        ],
        out_specs=[
            pl.BlockSpec((R, F2B), lambda i: (i, 0)),
            pl.BlockSpec((2, R), lambda i: (0, i)),
            pl.BlockSpec((2, 1), lambda i: (0, 0)),
        ],
        out_shape=[
            jax.ShapeDtypeStruct((N_PAD, F2B), jnp.bfloat16),
            jax.ShapeDtypeStruct((2, N_PAD), jnp.float32),
            jax.ShapeDtypeStruct((2, 1), jnp.float32),
        ],
    )(ufa, ufb, uda, udb, b1, w2p, alr2)


def _tc_final_body(ufa_ref, ufb_ref, uda_ref, udb_ref, b2_ref, wl_ref, bl_ref,
                   us_ref, sp_ref, scal_ref, out_ref):
    uf = (ufa_ref[...].astype(jnp.float32) + ufb_ref[...].astype(jnp.float32))
    den = uda_ref[:, 0:1] + udb_ref[:, 0:1]
    denom = jnp.maximum(den, 1e-9)
    x = _elu(uf / denom + b2_ref[...])
    zt = lax.dot_general(wl_ref[...], x, (((1,), (1,)), ((), ())),
                         preferred_element_type=jnp.float32)
    zt = zt + bl_ref[...]
    sig = 1.0 / (1.0 + jnp.exp(-zt))
    alpha0 = scal_ref[0:1, 0:1]
    beta0 = scal_ref[0:1, 1:2]
    gamma0 = scal_ref[0:1, 2:3]
    dt = scal_ref[0:1, 3:4]
    beta = sig[0:1, :] * beta0
    gamma = sig[1:2, :] * gamma0
    alphas = sig[2:3, :] * alpha0
    us = us_ref[...]
    sp = sp_ref[...]
    up_out = us + (alphas - beta * us) * dt
    sp_out = sp + (beta * us - gamma * sp) * dt
    zero3 = jnp.zeros((3, up_out.shape[1]), jnp.float32)
    out_ref[...] = jnp.concatenate([up_out, sp_out, alphas, beta, gamma, zero3], 0)


def _tc_final(ufa, ufb, uda, udb, b2, wlp, blp, us, sp, scal):
    grid = (N_PAD // R,)
    return pl.pallas_call(
        _tc_final_body,
        grid=grid,
        in_specs=[
            pl.BlockSpec((R, F2B), lambda i: (i, 0)),
            pl.BlockSpec((R, F2B), lambda i: (i, 0)),
            pl.BlockSpec((R, DW), lambda i: (i, 0)),
            pl.BlockSpec((R, DW), lambda i: (i, 0)),
            pl.BlockSpec((1, H2), lambda i: (0, 0)),
            pl.BlockSpec((8, H2), lambda i: (0, 0)),
            pl.BlockSpec((8, 1), lambda i: (0, 0)),
            pl.BlockSpec((1, R), lambda i: (0, i)),
            pl.BlockSpec((1, R), lambda i: (0, i)),
            pl.BlockSpec((1, 4), lambda i: (0, 0)),
        ],
        out_specs=[pl.BlockSpec((8, R), lambda i: (0, i))],
        out_shape=[jax.ShapeDtypeStruct((8, N_PAD), jnp.float32)],
    )(ufa, ufb, uda, udb, b2, wlp, blp, us, sp, scal)


_SC_PARAMS = pltpu.CompilerParams(
    needs_layout_passes=False, use_tc_tiling_on_sc=False)


def _sc_attention(eler_hbm, mvec_hbm, srcp_hbm, dstp_hbm, ee_hbm,
                  el_v, er_v, src_v, dst_v, m_v, ee_v):
    cid = lax.axis_index("c")
    sid = lax.axis_index("s")
    wid = sid * NC + cid
    pltpu.sync_copy(eler_hbm.at[0], el_v)
    pltpu.sync_copy(eler_hbm.at[1], er_v)
    pltpu.sync_copy(mvec_hbm, m_v)
    pltpu.sync_copy(srcp_hbm.at[wid], src_v)
    pltpu.sync_copy(dstp_hbm.at[wid], dst_v)
    mvec = m_v[...]

    def step(t, _):
        for j in range(4):
            o = t * 64 + j * LANES
            sv = src_v[pl.ds(o, LANES)]
            dv = dst_v[pl.ds(o, LANES)]
            elg = plsc.load_gather(el_v, [sv])
            erg = plsc.load_gather(er_v, [dv])
            x = elg + erg
            e = jnp.where(x >= 0, x, x * 0.2)
            ee_v[pl.ds(o, LANES)] = jnp.exp(e - mvec)
        return 0
    lax.fori_loop(0, EPT1 // 64, step, 0)
    pltpu.sync_copy(ee_v, ee_hbm.at[wid])


def _make_sc_attention():
    mesh = plsc.VectorSubcoreMesh(core_axis_name="c", subcore_axis_name="s")
    return functools.partial(
        pl.kernel,
        out_type=[jax.ShapeDtypeStruct((NW, EPT1), jnp.float32)],
        mesh=mesh,
        compiler_params=_SC_PARAMS,
        scratch_types=[
            pltpu.VMEM((N_PAD,), jnp.float32),
            pltpu.VMEM((N_PAD,), jnp.float32),
            pltpu.VMEM((EPT1,), jnp.int32),
            pltpu.VMEM((EPT1,), jnp.int32),
            pltpu.VMEM((LANES,), jnp.float32),
            pltpu.VMEM((EPT1,), jnp.float32),
        ],
    )(_sc_attention)


def _zero_accumulators(uf_sh, ud_sh, zf_v, zd_v, sid, ftw, zsem):
    zrow16 = jnp.zeros((2 * LANES,), jnp.bfloat16)
    zrow = jnp.zeros((LANES,), jnp.float32)

    def zb_row(zi, _):
        def zcol(qi, _):
            zf_v[zi, pl.ds(qi * 2 * LANES, 2 * LANES)] = zrow16
            return 0
        lax.fori_loop(0, ftw // (2 * LANES), zcol, 0)
        zd_v[zi, pl.ds(0, LANES)] = zrow
        return 0
    lax.fori_loop(0, ZROWS, zb_row, 0)

    nz = RPT // ZROWS
    def zdma(ji, _):
        pltpu.async_copy(zf_v, uf_sh.at[pl.ds(sid * RPT + ji * ZROWS, ZROWS)],
                         zsem)
        pltpu.async_copy(zd_v, ud_sh.at[pl.ds(sid * RPT + ji * ZROWS, ZROWS)],
                         zsem)
        return 0
    lax.fori_loop(0, nz, zdma, 0)
    return nz


def _zero_drain(uf_sh, ud_sh, zf_v, zd_v, sid, nz, zsem):
    def zdrain(ji, _):
        pltpu.make_async_copy(
            zf_v, uf_sh.at[pl.ds(sid * RPT, ZROWS)], zsem).wait()
        pltpu.make_async_copy(
            zd_v, ud_sh.at[pl.ds(sid * RPT, ZROWS)], zsem).wait()
        return 0
    lax.fori_loop(0, nz, zdrain, 0)


def _unpack_scale_pack(bfbuf, bfout, esp, ri, ee, ftw):
    mask_hi = jnp.full((LANES,), -65536, jnp.int32)
    for blk in range(ftw // 32):
        v = bfbuf[ri, pl.ds(blk * 32, 32)]
        w = plsc.bitcast(v, jnp.int32)
        lo = plsc.bitcast(lax.shift_left(w, 16), jnp.float32) * ee
        hi = plsc.bitcast(jnp.bitwise_and(w, mask_hi), jnp.float32) * ee
        bfout[ri, pl.ds(blk * 32, 2 * LANES)] = plsc.pack(
            lo, hi, format=plsc.PackFormat.INTERLEAVED)
    esp[ri, pl.ds(0, LANES)] = ee


def _make_sc_scatter1():
    mesh = plsc.VectorSubcoreMesh(core_axis_name="c", subcore_axis_name="s")
    ftw, k, cpt = F1B, K1, CPT1

    def body(hx_hbm, ee_hbm, srcp_hbm, dstp_hbm, outf_hbm, outd_hbm,
             src_v, dst_v, ee_v, bf0_v, bf1_v, fo0_v, fo1_v, es0_v, es1_v,
             zf_v, zd_v, uf_sh, ud_sh, gsem0, gsem1, ssem0, ssem1, zsem):
        cid = lax.axis_index("c")
        sid = lax.axis_index("s")
        wid = sid * NC + cid
        bf_bufs = (bf0_v, bf1_v)
        fo_bufs = (fo0_v, fo1_v)
        es_bufs = (es0_v, es1_v)
        gsems = (gsem0, gsem1)
        ssems = (ssem0, ssem1)

        nz = _zero_accumulators(uf_sh, ud_sh, zf_v, zd_v, sid, ftw, zsem)
        pltpu.sync_copy(srcp_hbm.at[wid], src_v)
        pltpu.sync_copy(dstp_hbm.at[wid], dst_v)
        pltpu.sync_copy(ee_hbm.at[wid], ee_v)
        _zero_drain(uf_sh, ud_sh, zf_v, zd_v, sid, nz, zsem)
        plsc.subcore_barrier()

        def gissue(ci, b):
            pltpu.async_copy(hx_hbm.at[src_v.at[ci]], bf_bufs[b], gsems[b])

        def gwait(b):
            pltpu.make_async_copy(hx_hbm.at[src_v.at[0]], bf_bufs[b],
                                  gsems[b]).wait()

        def sissue(ci, b):
            pltpu.async_copy(fo_bufs[b], uf_sh.at[dst_v.at[ci]], ssems[b],
                             add=True)
            pltpu.async_copy(es_bufs[b], ud_sh.at[dst_v.at[ci]], ssems[b],
                             add=True)

        def swait(b):
            pltpu.make_async_copy(fo_bufs[b], uf_sh.at[dst_v.at[0]],
                                  ssems[b]).wait()
            pltpu.make_async_copy(es_bufs[b], ud_sh.at[dst_v.at[0]],
                                  ssems[b]).wait()

        gissue(0, 0)

        def outer(g, _):
            for b in range(NBUF):
                c = g * NBUF + b

                @pl.when(c + 1 < cpt)
                def _():
                    gissue(c + 1, b ^ 1)
                gwait(b)

                @pl.when(c >= 2)
                def _():
                    swait(b)

                def scale4(rg, _):
                    base = c * k + rg * 4
                    for rr in range(4):
                        ri = rg * 4 + rr
                        ee = plsc.load_gather(
                            ee_v, [jnp.full((LANES,), base + rr, jnp.int32)])
                        _unpack_scale_pack(bf_bufs[b], fo_bufs[b], es_bufs[b],
                                           ri, ee, ftw)
                    return 0
                lax.fori_loop(0, k // 4, scale4, 0)
                sissue(c, b)
            return 0
        lax.fori_loop(0, cpt // NBUF, outer, 0)
        swait(0)
        swait(1)

        plsc.subcore_barrier()
        pltpu.sync_copy(uf_sh.at[pl.ds(sid * RPT, RPT)],
                        outf_hbm.at[cid, pl.ds(sid * RPT, RPT)])
        pltpu.sync_copy(ud_sh.at[pl.ds(sid * RPT, RPT)],
                        outd_hbm.at[cid, pl.ds(sid * RPT, RPT)])

    return functools.partial(
        pl.kernel,
        out_type=[jax.ShapeDtypeStruct((NC, N_PAD, ftw), jnp.bfloat16),
                  jax.ShapeDtypeStruct((NC, N_PAD, DW), jnp.float32)],
        mesh=mesh,
        compiler_params=_SC_PARAMS,
        scratch_types=[
            pltpu.VMEM((cpt, k), jnp.int32),
            pltpu.VMEM((cpt, k), jnp.int32),
            pltpu.VMEM((EPT1,), jnp.float32),
            pltpu.VMEM((k, ftw), jnp.bfloat16),
            pltpu.VMEM((k, ftw), jnp.bfloat16),
            pltpu.VMEM((k, ftw), jnp.bfloat16),
            pltpu.VMEM((k, ftw), jnp.bfloat16),
            pltpu.VMEM((k, DW), jnp.float32),
            pltpu.VMEM((k, DW), jnp.float32),
            pltpu.VMEM((ZROWS, ftw), jnp.bfloat16),
            pltpu.VMEM((ZROWS, DW), jnp.float32),
            pltpu.VMEM_SHARED((N_UPAD, ftw), jnp.bfloat16),
            pltpu.VMEM_SHARED((N_UPAD, DW), jnp.float32),
            pltpu.SemaphoreType.DMA,
            pltpu.SemaphoreType.DMA,
            pltpu.SemaphoreType.DMA,
            pltpu.SemaphoreType.DMA,
            pltpu.SemaphoreType.DMA,
        ],
    )(body)


def _make_sc_edge2():
    mesh = plsc.VectorSubcoreMesh(core_axis_name="c", subcore_axis_name="s")
    ftw, k, cpt = F2B, K2, CPT2

    def body(hx_hbm, eler_hbm, mvec_hbm, srcp_hbm, dstp_hbm, outf_hbm,
             outd_hbm, el_v, er_v, src_v, dst_v, m_v, ee_v, bf0_v, bf1_v,
             fo0_v, fo1_v, es0_v, es1_v, zf_v, zd_v, uf_sh, ud_sh,
             gsem0, gsem1, ssem0, ssem1, zsem):
        cid = lax.axis_index("c")
        sid = lax.axis_index("s")
        wid = sid * NC + cid
        bf_bufs = (bf0_v, bf1_v)
        fo_bufs = (fo0_v, fo1_v)
        es_bufs = (es0_v, es1_v)
        gsems = (gsem0, gsem1)
        ssems = (ssem0, ssem1)

        nz = _zero_accumulators(uf_sh, ud_sh, zf_v, zd_v, sid, ftw, zsem)
        pltpu.sync_copy(eler_hbm.at[0], el_v)
        pltpu.sync_copy(eler_hbm.at[1], er_v)
        pltpu.sync_copy(mvec_hbm, m_v)
        pltpu.sync_copy(srcp_hbm.at[wid], src_v)
        pltpu.sync_copy(dstp_hbm.at[wid], dst_v)
        _zero_drain(uf_sh, ud_sh, zf_v, zd_v, sid, nz, zsem)
        plsc.subcore_barrier()
        mvec = m_v[...]

        def gissue(ci, b):
            pltpu.async_copy(hx_hbm.at[src_v.at[ci]], bf_bufs[b], gsems[b])

        def gwait(b):
            pltpu.make_async_copy(hx_hbm.at[src_v.at[0]], bf_bufs[b],
                                  gsems[b]).wait()

        def sissue(ci, b):
            pltpu.async_copy(fo_bufs[b], uf_sh.at[dst_v.at[ci]], ssems[b],
                             add=True)
            pltpu.async_copy(es_bufs[b], ud_sh.at[dst_v.at[ci]], ssems[b],
                             add=True)

        def swait(b):
            pltpu.make_async_copy(fo_bufs[b], uf_sh.at[dst_v.at[0]],
                                  ssems[b]).wait()
            pltpu.make_async_copy(es_bufs[b], ud_sh.at[dst_v.at[0]],
                                  ssems[b]).wait()

        gissue(0, 0)

        def outer(g, _):
            for b in range(NBUF):
                c = g * NBUF + b

                @pl.when(c + 1 < cpt)
                def _():
                    gissue(c + 1, b ^ 1)

                for j in range(k // LANES):
                    sv = src_v[c, pl.ds(j * LANES, LANES)]
                    dv = dst_v[c, pl.ds(j * LANES, LANES)]
                    elg = plsc.load_gather(el_v, [sv])
                    erg = plsc.load_gather(er_v, [dv])
                    x = elg + erg
                    e = jnp.where(x >= 0, x, x * 0.2)
                    ee_v[pl.ds(j * LANES, LANES)] = jnp.exp(e - mvec)
                gwait(b)

                @pl.when(c >= 2)
                def _():
                    swait(b)

                def scale4(rg, _):
                    for rr in range(4):
                        ri = rg * 4 + rr
                        ee = plsc.load_gather(
                            ee_v, [jnp.full((LANES,), ri, jnp.int32)])
                        _unpack_scale_pack(bf_bufs[b], fo_bufs[b], es_bufs[b],
                                           ri, ee, ftw)
                    return 0
                lax.fori_loop(0, k // 4, scale4, 0)
                sissue(c, b)
            return 0
        lax.fori_loop(0, cpt // NBUF, outer, 0)
        swait(0)
        swait(1)

        plsc.subcore_barrier()
        pltpu.sync_copy(uf_sh.at[pl.ds(sid * RPT, RPT)],
                        outf_hbm.at[cid, pl.ds(sid * RPT, RPT)])
        pltpu.sync_copy(ud_sh.at[pl.ds(sid * RPT, RPT)],
                        outd_hbm.at[cid, pl.ds(sid * RPT, RPT)])

    return functools.partial(
        pl.kernel,
        out_type=[jax.ShapeDtypeStruct((NC, N_PAD, ftw), jnp.bfloat16),
                  jax.ShapeDtypeStruct((NC, N_PAD, DW), jnp.float32)],
        mesh=mesh,
        compiler_params=_SC_PARAMS,
        scratch_types=[
            pltpu.VMEM((N_PAD,), jnp.float32),
            pltpu.VMEM((N_PAD,), jnp.float32),
            pltpu.VMEM((cpt, k), jnp.int32),
            pltpu.VMEM((cpt, k), jnp.int32),
            pltpu.VMEM((LANES,), jnp.float32),
            pltpu.VMEM((k,), jnp.float32),
            pltpu.VMEM((k, ftw), jnp.bfloat16),
            pltpu.VMEM((k, ftw), jnp.bfloat16),
            pltpu.VMEM((k, ftw), jnp.bfloat16),
            pltpu.VMEM((k, ftw), jnp.bfloat16),
            pltpu.VMEM((k, DW), jnp.float32),
            pltpu.VMEM((k, DW), jnp.float32),
            pltpu.VMEM((ZROWS, ftw), jnp.bfloat16),
            pltpu.VMEM((ZROWS, DW), jnp.float32),
            pltpu.VMEM_SHARED((N_UPAD, ftw), jnp.bfloat16),
            pltpu.VMEM_SHARED((N_UPAD, DW), jnp.float32),
            pltpu.SemaphoreType.DMA,
            pltpu.SemaphoreType.DMA,
            pltpu.SemaphoreType.DMA,
            pltpu.SemaphoreType.DMA,
            pltpu.SemaphoreType.DMA,
        ],
    )(body)


_sc_att_1 = _make_sc_attention()
_sc_scatter_1 = _make_sc_scatter1()
_sc_edge_2 = _make_sc_edge2()


def _permute_rows(w_ext, al_ext, c_of_p, fbf):
    ncols = w_ext.shape[0]
    sel = np.nonzero(c_of_p < ncols)[0]
    csel = c_of_p[sel]
    wp = jnp.zeros((fbf, w_ext.shape[1]), jnp.float32).at[sel].set(w_ext[csel])
    alp = jnp.zeros((2, fbf), jnp.float32).at[:, sel].set(al_ext[:, csel])
    return wp, alp


def kernel(edge_index, feat, unsplice, splice, alpha0, beta0, gamma0, dt,
           W1, b1, al1, ar1, W2, b2, al2, ar2, Wl, bl):
    f32 = jnp.float32
    src = edge_index[0]
    dst = edge_index[1]
    src_p1 = jnp.concatenate([src, jnp.zeros((E_PAD1 - E,), jnp.int32)])
    dst_p1 = jnp.concatenate([dst, jnp.full((E_PAD1 - E,), N, jnp.int32)])
    src_p2 = jnp.concatenate([src, jnp.zeros((E_PAD2 - E,), jnp.int32)])
    dst_p2 = jnp.concatenate([dst, jnp.full((E_PAD2 - E,), N, jnp.int32)])
    srcp1 = src_p1.reshape(NW, CPT1, K1)
    dstp1 = dst_p1.reshape(NW, CPT1, K1)
    srcp1f = src_p1.reshape(NW, EPT1)
    dstp1f = dst_p1.reshape(NW, EPT1)
    srcp2 = src_p2.reshape(NW, CPT2, K2)
    dstp2 = dst_p2.reshape(NW, CPT2, K2)

    feat_p = jnp.zeros((N_PAD, IN_FEATS), f32).at[:N].set(feat)
    alr1e = jnp.concatenate([al1, ar1], 0)
    w1p, alr1 = _permute_rows(W1, alr1e, _C_OF_P1, F1B)
    alr2e = jnp.concatenate([al2, ar2], 0)
    w2p, alr2 = _permute_rows(W2, alr2e, _C_OF_P2, F2B)
    w2pp = w2p[:, _C_OF_P1]
    b1p = b1[_C_OF_P1].reshape(1, H1)
    wlpp = jnp.zeros((8, H2), f32).at[:3].set(Wl)[:, _C_OF_P2]
    b2p = b2[_C_OF_P2].reshape(1, H2)
    blp = jnp.zeros((8, 1), f32).at[:3, 0].set(bl)
    usp = jnp.zeros((1, N_PAD), f32).at[0, :N].set(unsplice)
    spp = jnp.zeros((1, N_PAD), f32).at[0, :N].set(splice)
    scal = jnp.stack([alpha0[0], beta0[0], gamma0[0], dt[0]]).reshape(1, 4)

    hx1, eler1, mm1 = _tc_pre(feat_p, w1p, alr1)
    mvec1 = jnp.full((LANES,), mm1[0, 0] + mm1[1, 0], f32)
    (ee1,) = _sc_att_1(eler1, mvec1, srcp1f, dstp1f)
    u1f, u1d = _sc_scatter_1(hx1, ee1, srcp1, dstp1)

    hx2, eler2, mm2 = _tc_mid(u1f[0], u1f[1], u1d[0], u1d[1], b1p, w2pp, alr2)
    mvec2 = jnp.full((LANES,), mm2[0, 0] + mm2[1, 0], f32)
    u2f, u2d = _sc_edge_2(hx2, eler2, mvec2, srcp2, dstp2)

    (out8,) = _tc_final(u2f[0], u2f[1], u2d[0], u2d[1], b2p, wlpp, blp,
                        usp, spp, scal)

    return (out8[0, :N], out8[1, :N], out8[2, :N], out8[3, :N], out8[4, :N])


# speedup vs baseline: 2.0878x; 1.0266x over previous
import functools

import numpy as np

import jax
import jax.numpy as jnp
from jax import lax
from jax.experimental import pallas as pl
from jax.experimental.pallas import tpu as pltpu
from jax.experimental.pallas import tpu_sc as plsc

N = 10000
E = 160000
IN_FEATS = 128
H1 = 128
H2 = 64

NC = 2
NS = 16
NW = NC * NS
LANES = 16

N_PAD = 10240
N_UPAD = 10048
R = 512
NBUF = 2
K1, CPT1 = 128, 40
K2, CPT2 = 128, 40
EPT1 = K1 * CPT1
EPT2 = K2 * CPT2
E_PAD1 = EPT1 * NW
E_PAD2 = EPT2 * NW

F1B = 128
F2B = 64
DW = 16
ZROWS = 4
RPT = N_UPAD // NS


def _unpack_order(fbf):
    c_of_p = np.zeros(fbf, dtype=np.int64)
    for p in range(fbf):
        j, t = divmod(p, 32)
        c_of_p[p] = 32 * j + (t // 2 if t % 2 == 0 else 16 + (t - 1) // 2)
    return c_of_p


_C_OF_P1 = _unpack_order(F1B)
_C_OF_P2 = _unpack_order(F2B)


def _elu(x):
    return jnp.where(x > 0, x, jnp.exp(jnp.minimum(x, 0.0)) - 1.0)


def _tc_pre_body(f_ref, w_ref, alr_ref, hx_ref, eler_ref, mm_ref):
    i = pl.program_id(0)
    h = lax.dot_general(f_ref[...], w_ref[...], (((1,), (1,)), ((), ())),
                        preferred_element_type=jnp.float32)
    hx_ref[...] = h.astype(jnp.bfloat16)
    eler = lax.dot_general(alr_ref[...], h, (((1,), (1,)), ((), ())),
                           preferred_element_type=jnp.float32)
    eler_ref[...] = eler
    mblk = jnp.max(eler, axis=1, keepdims=True)

    @pl.when(i == 0)
    def _():
        mm_ref[...] = mblk

    @pl.when(i > 0)
    def _():
        mm_ref[...] = jnp.maximum(mm_ref[...], mblk)


def _tc_pre(feat_p, w1p, alr1):
    grid = (N_PAD // R,)
    return pl.pallas_call(
        _tc_pre_body,
        grid=grid,
        in_specs=[
            pl.BlockSpec((R, IN_FEATS), lambda i: (i, 0)),
            pl.BlockSpec((F1B, IN_FEATS), lambda i: (0, 0)),
            pl.BlockSpec((2, F1B), lambda i: (0, 0)),
        ],
        out_specs=[
            pl.BlockSpec((R, F1B), lambda i: (i, 0)),
            pl.BlockSpec((2, R), lambda i: (0, i)),
            pl.BlockSpec((2, 1), lambda i: (0, 0)),
        ],
        out_shape=[
            jax.ShapeDtypeStruct((N_PAD, F1B), jnp.bfloat16),
            jax.ShapeDtypeStruct((2, N_PAD), jnp.float32),
            jax.ShapeDtypeStruct((2, 1), jnp.float32),
        ],
    )(feat_p, w1p, alr1)


def _tc_mid_body(ufa_ref, ufb_ref, uda_ref, udb_ref, b1_ref, w2_ref, alr_ref,
                 hx_ref, eler_ref, mm_ref):
    i = pl.program_id(0)
    uf = (ufa_ref[...].astype(jnp.float32) + ufb_ref[...].astype(jnp.float32))
    den = uda_ref[:, 0:1] + udb_ref[:, 0:1]
    denom = jnp.maximum(den, 1e-9)
    rst = uf / denom + b1_ref[...]
    x = _elu(_elu(rst))
    h = lax.dot_general(x, w2_ref[...], (((1,), (1,)), ((), ())),
                        preferred_element_type=jnp.float32)
    hx_ref[...] = h.astype(jnp.bfloat16)
    eler = lax.dot_general(alr_ref[...], h, (((1,), (1,)), ((), ())),
                           preferred_element_type=jnp.float32)
    row = lax.broadcasted_iota(jnp.int32, eler.shape, 1) + i * R
    eler = jnp.where(row < N_UPAD, eler, -1e30)
    eler_ref[...] = eler
    mblk = jnp.max(eler, axis=1, keepdims=True)

    @pl.when(i == 0)
    def _():
        mm_ref[...] = mblk

    @pl.when(i > 0)
    def _():
        mm_ref[...] = jnp.maximum(mm_ref[...], mblk)


def _tc_mid(ufa, ufb, uda, udb, b1, w2p, alr2):
    grid = (N_PAD // R,)
    return pl.pallas_call(
        _tc_mid_body,
        grid=grid,
        in_specs=[
            pl.BlockSpec((R, F1B), lambda i: (i, 0)),
            pl.BlockSpec((R, F1B), lambda i: (i, 0)),
            pl.BlockSpec((R, DW), lambda i: (i, 0)),
            pl.BlockSpec((R, DW), lambda i: (i, 0)),
            pl.BlockSpec((1, H1), lambda i: (0, 0)),
            pl.BlockSpec((F2B, H1), lambda i: (0, 0)),
            pl.BlockSpec((2, F2B), lambda i: (0, 0)),
        ],
        out_specs=[
            pl.BlockSpec((R, F2B), lambda i: (i, 0)),
            pl.BlockSpec((2, R), lambda i: (0, i)),
            pl.BlockSpec((2, 1), lambda i: (0, 0)),
        ],
        out_shape=[
            jax.ShapeDtypeStruct((N_PAD, F2B), jnp.bfloat16),
            jax.ShapeDtypeStruct((2, N_PAD), jnp.float32),
            jax.ShapeDtypeStruct((2, 1), jnp.float32),
        ],
    )(ufa, ufb, uda, udb, b1, w2p, alr2)


def _tc_final_body(ufa_ref, ufb_ref, uda_ref, udb_ref, b2_ref, wl_ref, bl_ref,
                   us_ref, sp_ref, scal_ref, out_ref):
    uf = (ufa_ref[...].astype(jnp.float32) + ufb_ref[...].astype(jnp.float32))
    den = uda_ref[:, 0:1] + udb_ref[:, 0:1]
    denom = jnp.maximum(den, 1e-9)
    x = _elu(uf / denom + b2_ref[...])
    zt = lax.dot_general(wl_ref[...], x, (((1,), (1,)), ((), ())),
                         preferred_element_type=jnp.float32)
    zt = zt + bl_ref[...]
    sig = 1.0 / (1.0 + jnp.exp(-zt))
    alpha0 = scal_ref[0:1, 0:1]
    beta0 = scal_ref[0:1, 1:2]
    gamma0 = scal_ref[0:1, 2:3]
    dt = scal_ref[0:1, 3:4]
    beta = sig[0:1, :] * beta0
    gamma = sig[1:2, :] * gamma0
    alphas = sig[2:3, :] * alpha0
    us = us_ref[...]
    sp = sp_ref[...]
    up_out = us + (alphas - beta * us) * dt
    sp_out = sp + (beta * us - gamma * sp) * dt
    zero3 = jnp.zeros((3, up_out.shape[1]), jnp.float32)
    out_ref[...] = jnp.concatenate([up_out, sp_out, alphas, beta, gamma, zero3], 0)


def _tc_final(ufa, ufb, uda, udb, b2, wlp, blp, us, sp, scal):
    grid = (N_PAD // R,)
    return pl.pallas_call(
        _tc_final_body,
        grid=grid,
        in_specs=[
            pl.BlockSpec((R, F2B), lambda i: (i, 0)),
            pl.BlockSpec((R, F2B), lambda i: (i, 0)),
            pl.BlockSpec((R, DW), lambda i: (i, 0)),
            pl.BlockSpec((R, DW), lambda i: (i, 0)),
            pl.BlockSpec((1, H2), lambda i: (0, 0)),
            pl.BlockSpec((8, H2), lambda i: (0, 0)),
            pl.BlockSpec((8, 1), lambda i: (0, 0)),
            pl.BlockSpec((1, R), lambda i: (0, i)),
            pl.BlockSpec((1, R), lambda i: (0, i)),
            pl.BlockSpec((1, 4), lambda i: (0, 0)),
        ],
        out_specs=[pl.BlockSpec((8, R), lambda i: (0, i))],
        out_shape=[jax.ShapeDtypeStruct((8, N_PAD), jnp.float32)],
    )(ufa, ufb, uda, udb, b2, wlp, blp, us, sp, scal)


_SC_PARAMS = pltpu.CompilerParams(
    needs_layout_passes=False, use_tc_tiling_on_sc=False)


def _sc_attention(eler_hbm, mvec_hbm, srcp_hbm, dstp_hbm, ee_hbm,
                  el_v, er_v, src_v, dst_v, m_v, ee_v):
    cid = lax.axis_index("c")
    sid = lax.axis_index("s")
    wid = sid * NC + cid
    pltpu.sync_copy(eler_hbm.at[0], el_v)
    pltpu.sync_copy(eler_hbm.at[1], er_v)
    pltpu.sync_copy(mvec_hbm, m_v)
    pltpu.sync_copy(srcp_hbm.at[wid], src_v)
    pltpu.sync_copy(dstp_hbm.at[wid], dst_v)
    mvec = m_v[...]

    def step(t, _):
        for j in range(4):
            o = t * 64 + j * LANES
            sv = src_v[pl.ds(o, LANES)]
            dv = dst_v[pl.ds(o, LANES)]
            elg = plsc.load_gather(el_v, [sv])
            erg = plsc.load_gather(er_v, [dv])
            x = elg + erg
            e = jnp.where(x >= 0, x, x * 0.2)
            ee_v[pl.ds(o, LANES)] = jnp.exp(e - mvec)
        return 0
    lax.fori_loop(0, EPT1 // 64, step, 0)
    pltpu.sync_copy(ee_v, ee_hbm.at[wid])


def _make_sc_attention():
    mesh = plsc.VectorSubcoreMesh(core_axis_name="c", subcore_axis_name="s")
    return functools.partial(
        pl.kernel,
        out_type=[jax.ShapeDtypeStruct((NW, EPT1), jnp.float32)],
        mesh=mesh,
        compiler_params=_SC_PARAMS,
        scratch_types=[
            pltpu.VMEM((N_PAD,), jnp.float32),
            pltpu.VMEM((N_PAD,), jnp.float32),
            pltpu.VMEM((EPT1,), jnp.int32),
            pltpu.VMEM((EPT1,), jnp.int32),
            pltpu.VMEM((LANES,), jnp.float32),
            pltpu.VMEM((EPT1,), jnp.float32),
        ],
    )(_sc_attention)


def _zero_accumulators(uf_sh, ud_sh, zf_v, zd_v, sid, ftw, zsem):
    zrow16 = jnp.zeros((2 * LANES,), jnp.bfloat16)
    zrow = jnp.zeros((LANES,), jnp.float32)

    def zb_row(zi, _):
        def zcol(qi, _):
            zf_v[zi, pl.ds(qi * 2 * LANES, 2 * LANES)] = zrow16
            return 0
        lax.fori_loop(0, ftw // (2 * LANES), zcol, 0)
        zd_v[zi, pl.ds(0, LANES)] = zrow
        return 0
    lax.fori_loop(0, ZROWS, zb_row, 0)

    nz = RPT // ZROWS
    def zdma(ji, _):
        pltpu.async_copy(zf_v, uf_sh.at[pl.ds(sid * RPT + ji * ZROWS, ZROWS)],
                         zsem)
        pltpu.async_copy(zd_v, ud_sh.at[pl.ds(sid * RPT + ji * ZROWS, ZROWS)],
                         zsem)
        return 0
    lax.fori_loop(0, nz, zdma, 0)
    return nz


def _zero_drain(uf_sh, ud_sh, zf_v, zd_v, sid, nz, zsem):
    def zdrain(ji, _):
        pltpu.make_async_copy(
            zf_v, uf_sh.at[pl.ds(sid * RPT, ZROWS)], zsem).wait()
        pltpu.make_async_copy(
            zd_v, ud_sh.at[pl.ds(sid * RPT, ZROWS)], zsem).wait()
        return 0
    lax.fori_loop(0, nz, zdrain, 0)


def _unpack_scale_pack(bfbuf, bfout, esp, ri, ee, ftw):
    mask_hi = jnp.full((LANES,), -65536, jnp.int32)
    for blk in range(ftw // 32):
        v = bfbuf[ri, pl.ds(blk * 32, 32)]
        w = plsc.bitcast(v, jnp.int32)
        lo = plsc.bitcast(lax.shift_left(w, 16), jnp.float32) * ee
        hi = plsc.bitcast(jnp.bitwise_and(w, mask_hi), jnp.float32) * ee
        bfout[ri, pl.ds(blk * 32, 2 * LANES)] = plsc.pack(
            lo, hi, format=plsc.PackFormat.INTERLEAVED)
    esp[ri, pl.ds(0, LANES)] = ee


def _make_sc_scatter1():
    mesh = plsc.VectorSubcoreMesh(core_axis_name="c", subcore_axis_name="s")
    ftw, k, cpt = F1B, K1, CPT1

    def body(hx_hbm, ee_hbm, srcp_hbm, dstp_hbm, outf0_hbm, outf1_hbm,
             outd0_hbm, outd1_hbm,
             src_v, dst_v, ee_v, bf0_v, bf1_v, fo0_v, fo1_v, es0_v, es1_v,
             zf_v, zd_v, uf_sh, ud_sh, gsem0, gsem1, ssem0, ssem1, zsem):
        cid = lax.axis_index("c")
        sid = lax.axis_index("s")
        wid = sid * NC + cid
        bf_bufs = (bf0_v, bf1_v)
        fo_bufs = (fo0_v, fo1_v)
        es_bufs = (es0_v, es1_v)
        gsems = (gsem0, gsem1)
        ssems = (ssem0, ssem1)

        nz = _zero_accumulators(uf_sh, ud_sh, zf_v, zd_v, sid, ftw, zsem)
        pltpu.sync_copy(srcp_hbm.at[wid], src_v)
        pltpu.sync_copy(dstp_hbm.at[wid], dst_v)
        pltpu.sync_copy(ee_hbm.at[wid], ee_v)
        _zero_drain(uf_sh, ud_sh, zf_v, zd_v, sid, nz, zsem)
        plsc.subcore_barrier()

        def gissue(ci, b):
            pltpu.async_copy(hx_hbm.at[src_v.at[ci]], bf_bufs[b], gsems[b])

        def gwait(b):
            pltpu.make_async_copy(hx_hbm.at[src_v.at[0]], bf_bufs[b],
                                  gsems[b]).wait()

        def sissue(ci, b):
            pltpu.async_copy(fo_bufs[b], uf_sh.at[dst_v.at[ci]], ssems[b],
                             add=True)
            pltpu.async_copy(es_bufs[b], ud_sh.at[dst_v.at[ci]], ssems[b],
                             add=True)

        def swait(b):
            pltpu.make_async_copy(fo_bufs[b], uf_sh.at[dst_v.at[0]],
                                  ssems[b]).wait()
            pltpu.make_async_copy(es_bufs[b], ud_sh.at[dst_v.at[0]],
                                  ssems[b]).wait()

        gissue(0, 0)

        def outer(g, _):
            for b in range(NBUF):
                c = g * NBUF + b

                @pl.when(c + 1 < cpt)
                def _():
                    gissue(c + 1, b ^ 1)
                gwait(b)

                @pl.when(c >= 2)
                def _():
                    swait(b)

                def scale4(rg, _):
                    base = c * k + rg * 4
                    for rr in range(4):
                        ri = rg * 4 + rr
                        ee = plsc.load_gather(
                            ee_v, [jnp.full((LANES,), base + rr, jnp.int32)])
                        _unpack_scale_pack(bf_bufs[b], fo_bufs[b], es_bufs[b],
                                           ri, ee, ftw)
                    return 0
                lax.fori_loop(0, k // 4, scale4, 0)
                sissue(c, b)
            return 0
        lax.fori_loop(0, cpt // NBUF, outer, 0)
        swait(0)
        swait(1)

        plsc.subcore_barrier()

        @pl.when(cid == 0)
        def _():
            pltpu.sync_copy(uf_sh.at[pl.ds(sid * RPT, RPT)],
                            outf0_hbm.at[pl.ds(sid * RPT, RPT)])
            pltpu.sync_copy(ud_sh.at[pl.ds(sid * RPT, RPT)],
                            outd0_hbm.at[pl.ds(sid * RPT, RPT)])

        @pl.when(cid == 1)
        def _():
            pltpu.sync_copy(uf_sh.at[pl.ds(sid * RPT, RPT)],
                            outf1_hbm.at[pl.ds(sid * RPT, RPT)])
            pltpu.sync_copy(ud_sh.at[pl.ds(sid * RPT, RPT)],
                            outd1_hbm.at[pl.ds(sid * RPT, RPT)])

    return functools.partial(
        pl.kernel,
        out_type=[jax.ShapeDtypeStruct((N_PAD, ftw), jnp.bfloat16),
                  jax.ShapeDtypeStruct((N_PAD, ftw), jnp.bfloat16),
                  jax.ShapeDtypeStruct((N_PAD, DW), jnp.float32),
                  jax.ShapeDtypeStruct((N_PAD, DW), jnp.float32)],
        mesh=mesh,
        compiler_params=_SC_PARAMS,
        scratch_types=[
            pltpu.VMEM((cpt, k), jnp.int32),
            pltpu.VMEM((cpt, k), jnp.int32),
            pltpu.VMEM((EPT1,), jnp.float32),
            pltpu.VMEM((k, ftw), jnp.bfloat16),
            pltpu.VMEM((k, ftw), jnp.bfloat16),
            pltpu.VMEM((k, ftw), jnp.bfloat16),
            pltpu.VMEM((k, ftw), jnp.bfloat16),
            pltpu.VMEM((k, DW), jnp.float32),
            pltpu.VMEM((k, DW), jnp.float32),
            pltpu.VMEM((ZROWS, ftw), jnp.bfloat16),
            pltpu.VMEM((ZROWS, DW), jnp.float32),
            pltpu.VMEM_SHARED((N_UPAD, ftw), jnp.bfloat16),
            pltpu.VMEM_SHARED((N_UPAD, DW), jnp.float32),
            pltpu.SemaphoreType.DMA,
            pltpu.SemaphoreType.DMA,
            pltpu.SemaphoreType.DMA,
            pltpu.SemaphoreType.DMA,
            pltpu.SemaphoreType.DMA,
        ],
    )(body)


def _make_sc_edge2():
    mesh = plsc.VectorSubcoreMesh(core_axis_name="c", subcore_axis_name="s")
    ftw, k, cpt = F2B, K2, CPT2

    def body(hx_hbm, eler_hbm, mvec_hbm, srcp_hbm, dstp_hbm, outf0_hbm,
             outf1_hbm, outd0_hbm, outd1_hbm,
             el_v, er_v, src_v, dst_v, m_v, ee_v, bf0_v, bf1_v,
             fo0_v, fo1_v, es0_v, es1_v, zf_v, zd_v, uf_sh, ud_sh,
             gsem0, gsem1, ssem0, ssem1, zsem):
        cid = lax.axis_index("c")
        sid = lax.axis_index("s")
        wid = sid * NC + cid
        bf_bufs = (bf0_v, bf1_v)
        fo_bufs = (fo0_v, fo1_v)
        es_bufs = (es0_v, es1_v)
        gsems = (gsem0, gsem1)
        ssems = (ssem0, ssem1)

        nz = _zero_accumulators(uf_sh, ud_sh, zf_v, zd_v, sid, ftw, zsem)
        pltpu.sync_copy(eler_hbm.at[0], el_v)
        pltpu.sync_copy(eler_hbm.at[1], er_v)
        pltpu.sync_copy(mvec_hbm, m_v)
        pltpu.sync_copy(srcp_hbm.at[wid], src_v)
        pltpu.sync_copy(dstp_hbm.at[wid], dst_v)
        _zero_drain(uf_sh, ud_sh, zf_v, zd_v, sid, nz, zsem)
        plsc.subcore_barrier()
        mvec = m_v[...]

        def gissue(ci, b):
            pltpu.async_copy(hx_hbm.at[src_v.at[ci]], bf_bufs[b], gsems[b])

        def gwait(b):
            pltpu.make_async_copy(hx_hbm.at[src_v.at[0]], bf_bufs[b],
                                  gsems[b]).wait()

        def sissue(ci, b):
            pltpu.async_copy(fo_bufs[b], uf_sh.at[dst_v.at[ci]], ssems[b],
                             add=True)
            pltpu.async_copy(es_bufs[b], ud_sh.at[dst_v.at[ci]], ssems[b],
                             add=True)

        def swait(b):
            pltpu.make_async_copy(fo_bufs[b], uf_sh.at[dst_v.at[0]],
                                  ssems[b]).wait()
            pltpu.make_async_copy(es_bufs[b], ud_sh.at[dst_v.at[0]],
                                  ssems[b]).wait()

        gissue(0, 0)

        def outer(g, _):
            for b in range(NBUF):
                c = g * NBUF + b

                @pl.when(c + 1 < cpt)
                def _():
                    gissue(c + 1, b ^ 1)

                for j in range(k // LANES):
                    sv = src_v[c, pl.ds(j * LANES, LANES)]
                    dv = dst_v[c, pl.ds(j * LANES, LANES)]
                    elg = plsc.load_gather(el_v, [sv])
                    erg = plsc.load_gather(er_v, [dv])
                    x = elg + erg
                    e = jnp.where(x >= 0, x, x * 0.2)
                    ee_v[pl.ds(j * LANES, LANES)] = jnp.exp(e - mvec)
                gwait(b)

                @pl.when(c >= 2)
                def _():
                    swait(b)

                def scale4(rg, _):
                    for rr in range(4):
                        ri = rg * 4 + rr
                        ee = plsc.load_gather(
                            ee_v, [jnp.full((LANES,), ri, jnp.int32)])
                        _unpack_scale_pack(bf_bufs[b], fo_bufs[b], es_bufs[b],
                                           ri, ee, ftw)
                    return 0
                lax.fori_loop(0, k // 4, scale4, 0)
                sissue(c, b)
            return 0
        lax.fori_loop(0, cpt // NBUF, outer, 0)
        swait(0)
        swait(1)

        plsc.subcore_barrier()

        @pl.when(cid == 0)
        def _():
            pltpu.sync_copy(uf_sh.at[pl.ds(sid * RPT, RPT)],
                            outf0_hbm.at[pl.ds(sid * RPT, RPT)])
            pltpu.sync_copy(ud_sh.at[pl.ds(sid * RPT, RPT)],
                            outd0_hbm.at[pl.ds(sid * RPT, RPT)])

        @pl.when(cid == 1)
        def _():
            pltpu.sync_copy(uf_sh.at[pl.ds(sid * RPT, RPT)],
                            outf1_hbm.at[pl.ds(sid * RPT, RPT)])
            pltpu.sync_copy(ud_sh.at[pl.ds(sid * RPT, RPT)],
                            outd1_hbm.at[pl.ds(sid * RPT, RPT)])

    return functools.partial(
        pl.kernel,
        out_type=[jax.ShapeDtypeStruct((N_PAD, ftw), jnp.bfloat16),
                  jax.ShapeDtypeStruct((N_PAD, ftw), jnp.bfloat16),
                  jax.ShapeDtypeStruct((N_PAD, DW), jnp.float32),
                  jax.ShapeDtypeStruct((N_PAD, DW), jnp.float32)],
        mesh=mesh,
        compiler_params=_SC_PARAMS,
        scratch_types=[
            pltpu.VMEM((N_PAD,), jnp.float32),
            pltpu.VMEM((N_PAD,), jnp.float32),
            pltpu.VMEM((cpt, k), jnp.int32),
            pltpu.VMEM((cpt, k), jnp.int32),
            pltpu.VMEM((LANES,), jnp.float32),
            pltpu.VMEM((k,), jnp.float32),
            pltpu.VMEM((k, ftw), jnp.bfloat16),
            pltpu.VMEM((k, ftw), jnp.bfloat16),
            pltpu.VMEM((k, ftw), jnp.bfloat16),
            pltpu.VMEM((k, ftw), jnp.bfloat16),
            pltpu.VMEM((k, DW), jnp.float32),
            pltpu.VMEM((k, DW), jnp.float32),
            pltpu.VMEM((ZROWS, ftw), jnp.bfloat16),
            pltpu.VMEM((ZROWS, DW), jnp.float32),
            pltpu.VMEM_SHARED((N_UPAD, ftw), jnp.bfloat16),
            pltpu.VMEM_SHARED((N_UPAD, DW), jnp.float32),
            pltpu.SemaphoreType.DMA,
            pltpu.SemaphoreType.DMA,
            pltpu.SemaphoreType.DMA,
            pltpu.SemaphoreType.DMA,
            pltpu.SemaphoreType.DMA,
        ],
    )(body)


_sc_att_1 = _make_sc_attention()
_sc_scatter_1 = _make_sc_scatter1()
_sc_edge_2 = _make_sc_edge2()


def _permute_rows(w_ext, al_ext, c_of_p, fbf):
    ncols = w_ext.shape[0]
    sel = np.nonzero(c_of_p < ncols)[0]
    csel = c_of_p[sel]
    wp = jnp.zeros((fbf, w_ext.shape[1]), jnp.float32).at[sel].set(w_ext[csel])
    alp = jnp.zeros((2, fbf), jnp.float32).at[:, sel].set(al_ext[:, csel])
    return wp, alp


def kernel(edge_index, feat, unsplice, splice, alpha0, beta0, gamma0, dt,
           W1, b1, al1, ar1, W2, b2, al2, ar2, Wl, bl):
    f32 = jnp.float32
    src = edge_index[0]
    dst = edge_index[1]
    src_p1 = jnp.concatenate([src, jnp.zeros((E_PAD1 - E,), jnp.int32)])
    dst_p1 = jnp.concatenate([dst, jnp.full((E_PAD1 - E,), N, jnp.int32)])
    src_p2 = jnp.concatenate([src, jnp.zeros((E_PAD2 - E,), jnp.int32)])
    dst_p2 = jnp.concatenate([dst, jnp.full((E_PAD2 - E,), N, jnp.int32)])
    srcp1 = src_p1.reshape(NW, CPT1, K1)
    dstp1 = dst_p1.reshape(NW, CPT1, K1)
    srcp1f = src_p1.reshape(NW, EPT1)
    dstp1f = dst_p1.reshape(NW, EPT1)
    srcp2 = src_p2.reshape(NW, CPT2, K2)
    dstp2 = dst_p2.reshape(NW, CPT2, K2)

    feat_p = jnp.zeros((N_PAD, IN_FEATS), f32).at[:N].set(feat)
    alr1e = jnp.concatenate([al1, ar1], 0)
    w1p, alr1 = _permute_rows(W1, alr1e, _C_OF_P1, F1B)
    alr2e = jnp.concatenate([al2, ar2], 0)
    w2p, alr2 = _permute_rows(W2, alr2e, _C_OF_P2, F2B)
    w2pp = w2p[:, _C_OF_P1]
    b1p = b1[_C_OF_P1].reshape(1, H1)
    wlpp = jnp.zeros((8, H2), f32).at[:3].set(Wl)[:, _C_OF_P2]
    b2p = b2[_C_OF_P2].reshape(1, H2)
    blp = jnp.zeros((8, 1), f32).at[:3, 0].set(bl)
    usp = jnp.zeros((1, N_PAD), f32).at[0, :N].set(unsplice)
    spp = jnp.zeros((1, N_PAD), f32).at[0, :N].set(splice)
    scal = jnp.stack([alpha0[0], beta0[0], gamma0[0], dt[0]]).reshape(1, 4)

    hx1, eler1, mm1 = _tc_pre(feat_p, w1p, alr1)
    mvec1 = jnp.full((LANES,), mm1[0, 0] + mm1[1, 0], f32)
    (ee1,) = _sc_att_1(eler1, mvec1, srcp1f, dstp1f)
    u1f0, u1f1, u1d0, u1d1 = _sc_scatter_1(hx1, ee1, srcp1, dstp1)

    hx2, eler2, mm2 = _tc_mid(u1f0, u1f1, u1d0, u1d1, b1p, w2pp, alr2)
    mvec2 = jnp.full((LANES,), mm2[0, 0] + mm2[1, 0], f32)
    u2f0, u2f1, u2d0, u2d1 = _sc_edge_2(hx2, eler2, mvec2, srcp2, dstp2)

    (out8,) = _tc_final(u2f0, u2f1, u2d0, u2d1, b2p, wlpp, blp,
                        usp, spp, scal)

    return (out8[0, :N], out8[1, :N], out8[2, :N], out8[3, :N], out8[4, :N])
